# baseline (speedup 1.0000x reference)
import jax
import jax.numpy as jnp
from jax import lax
from jax.experimental import pallas as pl
from jax.experimental.pallas import tpu as pltpu
from jax.experimental.pallas import tpu_sc as plsc

N = 10000
E = 160000
D = 128
H = 8
DH = 16
TEMP = 4.0

NCORE = 2
NSUB = 16
NW = NCORE * NSUB
LANES = 16

RPT = 320
NPAD = NW * RPT
RPC = NPAD // 2
ECH = E // NSUB
EW = 5120
EP2 = NW * EW
IDXW = 80
GW = 128
GW3 = 64
TA = 384
TB = 384
NCOMBO = 3200
W5 = 512

_mesh = plsc.VectorSubcoreMesh(core_axis_name="c", subcore_axis_name="s")
_SC_PARAMS = pltpu.CompilerParams(needs_layout_passes=False)


def _f32(shape):
    return jax.ShapeDtypeStruct(shape, jnp.float32)


def _i32(shape):
    return jax.ShapeDtypeStruct(shape, jnp.int32)


def _wid():
    return lax.axis_index("s") * NCORE + lax.axis_index("c")


def _mo(x, n=8):
    return pl.multiple_of(x, n)


def _zero_i32(ref, n):
    def body(i, _):
        ref[pl.ds(i * LANES, LANES)] = jnp.zeros((LANES,), jnp.int32)
        return 0
    lax.fori_loop(0, n // LANES, body, 0)


def _fill2d_f32(ref, rows, width, val):
    nv = width // LANES

    def body(i, _):
        ref[i // nv, pl.ds((i % nv) * LANES, LANES)] = jnp.full(
            (LANES,), val, jnp.float32)
        return 0
    lax.fori_loop(0, rows * nv, body, 0)


def _clamp_to_2d(src_flat, dst2d, total, width, lo, hi):
    nv = width // LANES

    def body(i, _):
        v = src_flat[pl.ds(i * LANES, LANES)]
        dst2d[i // nv, pl.ds((i % nv) * LANES, LANES)] = jnp.clip(v, lo, hi)
        return 0
    lax.fori_loop(0, total // LANES, body, 0)


def _k1_sort(row_hbm, col_hbm, rowS, colS, idxS, cnt_hbm, off_hbm, inv_hbm,
             rows_v, payl_v, rank_v, hist_v, tmp_v, base_v, tot_v,
             off_v, posj_v, rowj_v, colj_v, idxj_v, grid_sh):
    c = lax.axis_index("c")
    s = lax.axis_index("s")
    rlo = c * RPC
    eb = _mo(s * ECH)

    pltpu.sync_copy(row_hbm.at[pl.ds(eb, ECH)], rows_v)
    _zero_i32(hist_v, RPC)

    def hist_body(i, _):
        sl = pl.ds(_mo(i * LANES, LANES), LANES)
        r = rows_v[sl]
        m = (r >= rlo) & (r < rlo + RPC)
        rl = jnp.where(m, r - rlo, 0)
        occ, last = plsc.scan_count(rl, mask=m)
        base = plsc.load_gather(hist_v, [rl])
        rank_v[sl] = base + occ - 1
        plsc.store_scatter(hist_v, [rl], base + occ, mask=last & m)
        return 0
    lax.fori_loop(0, ECH // LANES, hist_body, 0)

    pltpu.sync_copy(hist_v, grid_sh.at[pl.ds(_mo(s * RPC), RPC)])
    plsc.subcore_barrier()

    _zero_i32(base_v, RPC)
    _zero_i32(tot_v, RPC)
    for sp in range(NSUB):
        pltpu.sync_copy(grid_sh.at[pl.ds(sp * RPC, RPC)], tmp_v)
        use = jnp.int32(sp) < s

        def acc_body(i, _):
            sl = pl.ds(_mo(i * LANES, LANES), LANES)
            tv = tmp_v[sl]
            base_v[sl] = base_v[sl] + jnp.where(use, tv, 0)
            tot_v[sl] = tot_v[sl] + tv
            return 0
        lax.fori_loop(0, RPC // LANES, acc_body, 0)

    def sum_body(i, acc):
        return acc + jnp.sum(tot_v[pl.ds(_mo(i * LANES, LANES), LANES)])
    core_total = lax.fori_loop(0, RPC // LANES, sum_body, jnp.int32(0))
    core_base = jnp.where(c == 0, jnp.int32(0), jnp.int32(E) - core_total)

    def scan_body(i, carry):
        sl = pl.ds(_mo(i * LANES, LANES), LANES)
        v = tot_v[sl]
        cs = plsc.cumsum(v)
        off_v[sl] = carry + core_base + cs - v
        return carry + jnp.sum(v)
    lax.fori_loop(0, RPC // LANES, scan_body, jnp.int32(0))

    sl = pl.ds(_mo(s * RPT), RPT)
    pltpu.sync_copy(tot_v.at[sl], cnt_hbm.at[pl.ds(_mo(c * RPC + s * RPT),
                                                   RPT)])
    pltpu.sync_copy(off_v.at[sl], off_hbm.at[pl.ds(_mo(c * RPC + s * RPT),
                                                   RPT)])

    pltpu.sync_copy(col_hbm.at[pl.ds(eb, ECH)], payl_v)

    def chunk(j, _):
        cb = _mo(j * IDXW)

        def vec(k2, _):
            sl_src = pl.ds(cb + _mo(k2 * LANES, LANES), LANES)
            sl_dst = pl.ds(_mo(k2 * LANES, LANES), LANES)
            r = rows_v[sl_src]
            m = (r >= rlo) & (r < rlo + RPC)
            rl = jnp.where(m, r - rlo, 0)
            pos = (plsc.load_gather(off_v, [rl])
                   + plsc.load_gather(base_v, [rl])
                   + rank_v[sl_src])
            pos = jnp.where(m, pos,
                            jnp.int32(E) + lax.iota(jnp.int32, LANES))
            posj_v[sl_dst] = pos
            rowj_v[sl_dst] = r
            colj_v[sl_dst] = payl_v[sl_src]
            idxj_v[sl_dst] = jnp.where(
                m, eb + cb + k2 * LANES + lax.iota(jnp.int32, LANES),
                jnp.int32(E) + lax.iota(jnp.int32, LANES))
            return 0
        lax.fori_loop(0, IDXW // LANES, vec, 0)
        pltpu.sync_copy(rowj_v, rowS.at[posj_v])
        pltpu.sync_copy(colj_v, colS.at[posj_v])
        pltpu.sync_copy(idxj_v, idxS.at[posj_v])
        pltpu.sync_copy(posj_v, inv_hbm.at[idxj_v])
        return 0
    lax.fori_loop(0, ECH // IDXW, chunk, 0)


def _run_k1(row, col):
    kern = pl.kernel(
        _k1_sort,
        out_type=(_i32((EP2,)), _i32((EP2,)), _i32((EP2,)),
                  _i32((NPAD,)), _i32((NPAD,)), _i32((E + LANES,))),
        mesh=_mesh,
        compiler_params=_SC_PARAMS,
        scratch_types=[
            pltpu.VMEM((ECH,), jnp.int32),
            pltpu.VMEM((ECH,), jnp.int32),
            pltpu.VMEM((ECH,), jnp.int32),
            pltpu.VMEM((RPC,), jnp.int32),
            pltpu.VMEM((RPC,), jnp.int32),
            pltpu.VMEM((RPC,), jnp.int32),
            pltpu.VMEM((RPC,), jnp.int32),
            pltpu.VMEM((RPC,), jnp.int32),
            pltpu.VMEM((IDXW,), jnp.int32),
            pltpu.VMEM((IDXW,), jnp.int32),
            pltpu.VMEM((IDXW,), jnp.int32),
            pltpu.VMEM((IDXW,), jnp.int32),
            pltpu.VMEM_SHARED((NSUB * RPC,), jnp.int32),
        ],
        name="k1_sort",
    )
    return kern(row, col)


def _k2_walk(rowS_hbm, colS_hbm, off_hbm, cnt_hbm, idxS,
             ridx_hbm, match_hbm,
             tgt_v, ptr_v, end_v, hitp_v, vals_v, res_v, flj_v, sem):
    wid = _wid()
    eb = _mo(wid * EW)
    nch = EW // GW

    pltpu.sync_copy(rowS_hbm.at[pl.ds(eb, EW)], tgt_v)
    pltpu.sync_copy(colS_hbm.at[pl.ds(eb, EW)], vals_v)

    def g_off(j, _):
        cb = _mo(j * GW)

        def cl(i2, _):
            v = vals_v[pl.ds(cb + _mo(i2 * LANES, LANES), LANES)]
            flj_v[pl.ds(_mo(i2 * LANES, LANES), LANES)] = jnp.clip(
                v, 0, NPAD - 1)
            return 0
        lax.fori_loop(0, GW // LANES, cl, 0)
        pltpu.async_copy(off_hbm.at[flj_v], ptr_v.at[pl.ds(cb, GW)],
                         sem).wait()
        pltpu.async_copy(cnt_hbm.at[flj_v], end_v.at[pl.ds(cb, GW)],
                         sem).wait()
        return 0
    lax.fori_loop(0, nch, g_off, 0)

    def init_body(i, _):
        sl = pl.ds(_mo(i * LANES, LANES), LANES)
        end_v[sl] = ptr_v[sl] + end_v[sl]
        hitp_v[sl] = jnp.full((LANES,), -1, jnp.int32)
        return 0
    lax.fori_loop(0, EW // LANES, init_body, 0)

    def walk_cond(carry):
        return carry > 0

    def walk_body(carry):
        def g_vals(j, _):
            cb = _mo(j * GW)

            def cl(i2, _):
                p = ptr_v[pl.ds(cb + _mo(i2 * LANES, LANES), LANES)]
                ok = (p >= 0) & (p < E)
                flj_v[pl.ds(_mo(i2 * LANES, LANES), LANES)] = jnp.where(
                    ok, p, E)
                return 0
            lax.fori_loop(0, GW // LANES, cl, 0)
            pltpu.async_copy(colS_hbm.at[flj_v],
                             vals_v.at[pl.ds(cb, GW)], sem).wait()

            def upd(i2, n_act):
                sl = pl.ds(cb + _mo(i2 * LANES, LANES), LANES)
                p = ptr_v[sl]
                active = (p < end_v[sl]) & (hitp_v[sl] < 0)
                hit = active & (vals_v[sl] == tgt_v[sl])
                hitp_v[sl] = jnp.where(hit, p, hitp_v[sl])
                still = active & ~hit
                ptr_v[sl] = jnp.where(still, p + 1, p)
                return n_act + jnp.sum(jnp.where(still, 1, 0))
            return lax.fori_loop(0, GW // LANES, upd, jnp.int32(0))
        return lax.fori_loop(0, nch, g_vals, jnp.int32(0))

    lax.while_loop(walk_cond, walk_body, jnp.int32(1))

    def g_res(j, _):
        cb = _mo(j * GW)

        def cl(i2, _):
            p = hitp_v[pl.ds(cb + _mo(i2 * LANES, LANES), LANES)]
            flj_v[pl.ds(_mo(i2 * LANES, LANES), LANES)] = jnp.where(
                p >= 0, p, E)
            return 0
        lax.fori_loop(0, GW // LANES, cl, 0)
        pltpu.async_copy(idxS.at[flj_v], vals_v.at[pl.ds(cb, GW)],
                         sem).wait()
        return 0
    lax.fori_loop(0, nch, g_res, 0)

    def fin(i, _):
        sl = pl.ds(_mo(i * LANES, LANES), LANES)
        ok = hitp_v[sl] >= 0
        res_v[sl] = jnp.clip(jnp.where(ok, vals_v[sl], 0), 0, E - 1)
        ptr_v[sl] = jnp.where(ok, 1, 0)
        return 0
    lax.fori_loop(0, EW // LANES, fin, 0)

    pltpu.sync_copy(res_v, ridx_hbm.at[pl.ds(eb, EW)])
    pltpu.sync_copy(ptr_v, match_hbm.at[pl.ds(eb, EW)])


def _run_k2(rowS, colS, off, cnt, idxS):
    kern = pl.kernel(
        _k2_walk,
        out_type=(_i32((EP2,)), _i32((EP2,))),
        mesh=_mesh,
        compiler_params=_SC_PARAMS,
        scratch_types=[
            pltpu.VMEM((EW,), jnp.int32),
            pltpu.VMEM((EW,), jnp.int32),
            pltpu.VMEM((EW,), jnp.int32),
            pltpu.VMEM((EW,), jnp.int32),
            pltpu.VMEM((EW,), jnp.int32),
            pltpu.VMEM((EW,), jnp.int32),
            pltpu.VMEM((GW,), jnp.int32),
            pltpu.SemaphoreType.DMA,
        ],
        name="k2_revlookup",
    )
    return kern(rowS, colS, off, cnt, idxS)


def _k3_gather(rowS_hbm, colS_hbm, idxS_hbm, obj_hbm, rel_hbm, ridx_hbm,
               row_tab, col_tab, pt_tab, ef_hbm,
               a_out, b_out, t_out, r_out, efs_out,
               rowf_v, colf_v, idxf_v, ridxf_v,
               rowj_v, colj_v, idxj_v, ridxj_v, comboj_v, objcj_v, relj_v,
               awin, bwin, twin, rwin, ewin, sem):
    wid = _wid()
    eb = _mo(wid * EW)
    nch = EW // GW3
    nv = GW3 // LANES

    def load_clamp(src_hbm, dst, hi):
        pltpu.sync_copy(src_hbm.at[pl.ds(eb, EW)], dst)

        def body(i, _):
            sl = pl.ds(_mo(i * LANES, LANES), LANES)
            dst[sl] = jnp.clip(dst[sl], 0, hi)
            return 0
        lax.fori_loop(0, EW // LANES, body, 0)

    load_clamp(rowS_hbm, rowf_v, N - 1)
    load_clamp(colS_hbm, colf_v, N - 1)
    load_clamp(idxS_hbm, idxf_v, E - 1)
    load_clamp(ridx_hbm, ridxf_v, E - 1)

    def chunk(j, _):
        cb = _mo(j * GW3)

        def cp(i2, _):
            sls = pl.ds(cb + _mo(i2 * LANES, LANES), LANES)
            sld = pl.ds(_mo(i2 * LANES, LANES), LANES)
            rowj_v[sld] = rowf_v[sls]
            colj_v[sld] = colf_v[sls]
            idxj_v[sld] = idxf_v[sls]
            ridxj_v[sld] = ridxf_v[sls]
            return 0
        lax.fori_loop(0, nv, cp, 0)

        pltpu.async_copy(obj_hbm.at[rowj_v], comboj_v, sem).wait()
        pltpu.async_copy(obj_hbm.at[colj_v], objcj_v, sem).wait()
        pltpu.async_copy(rel_hbm.at[idxj_v], relj_v, sem).wait()

        def mix(i2, _):
            sl = pl.ds(_mo(i2 * LANES, LANES), LANES)
            comboj_v[sl] = (comboj_v[sl] * 160 + relj_v[sl] * 20
                            + objcj_v[sl])
            return 0
        lax.fori_loop(0, nv, mix, 0)

        ob = pl.ds(eb + cb, GW3)
        pltpu.async_copy(row_tab.at[rowj_v], awin, sem).wait()
        pltpu.sync_copy(awin, a_out.at[ob])
        pltpu.async_copy(col_tab.at[colj_v], bwin, sem).wait()
        pltpu.sync_copy(bwin, b_out.at[ob])
        pltpu.async_copy(pt_tab.at[comboj_v], twin, sem).wait()
        pltpu.sync_copy(twin, t_out.at[ob])
        pltpu.async_copy(ef_hbm.at[idxj_v], ewin, sem).wait()
        pltpu.sync_copy(ewin, efs_out.at[ob])
        pltpu.async_copy(ef_hbm.at[ridxj_v], rwin, sem).wait()
        pltpu.sync_copy(rwin, r_out.at[ob])
        return 0
    lax.fori_loop(0, nch, chunk, 0)


def _run_k3(rowS, colS, idxS, obj, rel, ridx, row_tab, col_tab, pt_tab,
            ef):
    kern = pl.kernel(
        _k3_gather,
        out_type=(_f32((EP2, TA)), _f32((EP2, TB)), _f32((EP2, D)),
                  _f32((EP2, D)), _f32((EP2, D))),
        mesh=_mesh,
        compiler_params=_SC_PARAMS,
        scratch_types=[
            pltpu.VMEM((EW,), jnp.int32),
            pltpu.VMEM((EW,), jnp.int32),
            pltpu.VMEM((EW,), jnp.int32),
            pltpu.VMEM((EW,), jnp.int32),
            pltpu.VMEM((GW3,), jnp.int32),
            pltpu.VMEM((GW3,), jnp.int32),
            pltpu.VMEM((GW3,), jnp.int32),
            pltpu.VMEM((GW3,), jnp.int32),
            pltpu.VMEM((GW3,), jnp.int32),
            pltpu.VMEM((GW3,), jnp.int32),
            pltpu.VMEM((GW3,), jnp.int32),
            pltpu.VMEM((GW3, TA), jnp.float32),
            pltpu.VMEM((GW3, TB), jnp.float32),
            pltpu.VMEM((GW3, D), jnp.float32),
            pltpu.VMEM((GW3, D), jnp.float32),
            pltpu.VMEM((GW3, D), jnp.float32),
            pltpu.SemaphoreType.DMA,
        ],
        name="k3_gather",
    )
    return kern(rowS, colS, idxS, obj, rel, ridx, row_tab, col_tab,
                pt_tab, ef)


def _k5a_ms(lpf_hbm, rowS_hbm, off_hbm, ms_out,
            m_tab, s_tab, ms_buf, rows_w, lpw, ob):
    wid = _wid()
    rlo = wid * RPT

    pltpu.sync_copy(off_hbm.at[pl.ds(_mo(rlo), LANES)], ob)
    start = ob[pl.ds(0, LANES)][0]
    is_last = wid == NW - 1
    nxt = _mo(jnp.where(is_last, NPAD - LANES, rlo + RPT))
    pltpu.sync_copy(off_hbm.at[pl.ds(nxt, LANES)], ob)
    end = jnp.where(is_last, jnp.int32(E), ob[pl.ds(0, LANES)][0])

    abase = start - lax.rem(start, jnp.int32(8))
    nwin = (end - abase + (W5 - 1)) // W5

    def fill(ref, n, val):
        def body(i, _):
            ref[pl.ds(_mo(i * LANES, LANES), LANES)] = jnp.full(
                (LANES,), val, jnp.float32)
            return 0
        lax.fori_loop(0, n // LANES, body, 0)

    fill(m_tab, RPT * LANES, -jnp.inf)
    fill(s_tab, RPT * LANES, 0.0)

    def win_common(w):
        base_u = abase + w * W5
        base = _mo(jnp.minimum(base_u, jnp.int32(E - W5)))
        pltpu.sync_copy(lpf_hbm.at[pl.ds(_mo(base * 32), W5 * 32)], lpw)
        pltpu.sync_copy(rowS_hbm.at[pl.ds(base, W5)], rows_w)
        return base_u, base

    def win_a(w, _):
        base_u, base = win_common(w)

        def edge(i16, _):
            rvec = rows_w[pl.ds(_mo(i16 * LANES, LANES), LANES)]
            for k in range(LANES):
                i = i16 * LANES + k
                rloc = rvec[k] - rlo
                pp = base + i
                ok = ((rloc >= 0) & (rloc < RPT) & (pp >= start)
                      & (pp < end) & (pp >= base_u))
                rc = jnp.clip(rloc, 0, RPT - 1)
                lv = lpw[pl.ds(_mo(i * 32, LANES), LANES)]
                msl = pl.ds(_mo(rc * LANES, LANES), LANES)
                cur = m_tab[msl]
                m_tab[msl] = jnp.where(ok, jnp.maximum(cur, lv), cur)
            return 0
        lax.fori_loop(0, W5 // LANES, edge, 0)
        return 0
    lax.fori_loop(0, nwin, win_a, 0)

    def win_b(w, _):
        base_u, base = win_common(w)

        def edge(i16, _):
            rvec = rows_w[pl.ds(_mo(i16 * LANES, LANES), LANES)]
            for k in range(LANES):
                i = i16 * LANES + k
                rloc = rvec[k] - rlo
                pp = base + i
                ok = ((rloc >= 0) & (rloc < RPT) & (pp >= start)
                      & (pp < end) & (pp >= base_u))
                rc = jnp.clip(rloc, 0, RPT - 1)
                lv = lpw[pl.ds(_mo(i * 32, LANES), LANES)]
                msl = pl.ds(_mo(rc * LANES, LANES), LANES)
                ev = jnp.exp(lv - m_tab[msl])
                s_tab[msl] = s_tab[msl] + jnp.where(ok, ev, 0.0)
            return 0
        lax.fori_loop(0, W5 // LANES, edge, 0)
        return 0
    lax.fori_loop(0, nwin, win_b, 0)

    fill(ms_buf, RPT * D, 0.0)

    def pack(r, _):
        sl = pl.ds(_mo(r * LANES, LANES), LANES)
        ms_buf[pl.ds(_mo(r * D), LANES)] = m_tab[sl]
        ms_buf[pl.ds(_mo(r * D + LANES), LANES)] = s_tab[sl]
        return 0
    lax.fori_loop(0, RPT, pack, 0)

    pltpu.sync_copy(ms_buf, ms_out.at[pl.ds(_mo(rlo * D), RPT * D)])


def _run_k5a(lpf, rowS, off):
    kern = pl.kernel(
        _k5a_ms,
        out_type=_f32((NPAD * D,)),
        mesh=_mesh,
        compiler_params=_SC_PARAMS,
        scratch_types=[
            pltpu.VMEM((RPT * LANES,), jnp.float32),
            pltpu.VMEM((RPT * LANES,), jnp.float32),
            pltpu.VMEM((RPT * D,), jnp.float32),
            pltpu.VMEM((W5,), jnp.int32),
            pltpu.VMEM((W5 * 32,), jnp.float32),
            pltpu.VMEM((LANES,), jnp.int32),
        ],
        name="k5a_softmax_stats",
    )
    return kern(lpf, rowS, off)


NEGBIG = -3.4e38


def _k5b_agg(msgf_hbm, rowS_hbm, off_hbm, agg_out,
             agg_tab, rows_w, mw, ob):
    wid = _wid()
    rlo = wid * RPT

    pltpu.sync_copy(off_hbm.at[pl.ds(_mo(rlo), LANES)], ob)
    start = ob[pl.ds(0, LANES)][0]
    is_last = wid == NW - 1
    nxt = _mo(jnp.where(is_last, NPAD - LANES, rlo + RPT))
    pltpu.sync_copy(off_hbm.at[pl.ds(nxt, LANES)], ob)
    end = jnp.where(is_last, jnp.int32(E), ob[pl.ds(0, LANES)][0])

    abase = start - lax.rem(start, jnp.int32(8))
    nwin = (end - abase + (W5 - 1)) // W5

    def fill(ref, n, val):
        def body(i, _):
            ref[pl.ds(_mo(i * LANES, LANES), LANES)] = jnp.full(
                (LANES,), val, jnp.float32)
            return 0
        lax.fori_loop(0, n // LANES, body, 0)

    fill(agg_tab, RPT * D, NEGBIG)

    def win(w, _):
        base_u = abase + w * W5
        base = _mo(jnp.minimum(base_u, jnp.int32(E - W5)))
        pltpu.sync_copy(msgf_hbm.at[pl.ds(_mo(base * D), W5 * D)], mw)
        pltpu.sync_copy(rowS_hbm.at[pl.ds(base, W5)], rows_w)

        def edge(i16, _):
            rvec = rows_w[pl.ds(_mo(i16 * LANES, LANES), LANES)]
            for k in range(LANES):
                i = i16 * LANES + k
                rloc = rvec[k] - rlo
                pp = base + i
                ok = ((rloc >= 0) & (rloc < RPT) & (pp >= start)
                      & (pp < end) & (pp >= base_u))
                rc = jnp.clip(rloc, 0, RPT - 1)
                for h in range(D // LANES):
                    asl = pl.ds(_mo(rc * D + h * LANES, LANES), LANES)
                    cur = agg_tab[asl]
                    v = mw[pl.ds(_mo(i * D + h * LANES, LANES), LANES)]
                    agg_tab[asl] = jnp.where(ok, jnp.maximum(cur, v), cur)
            return 0
        lax.fori_loop(0, W5 // LANES, edge, 0)
        return 0
    lax.fori_loop(0, nwin, win, 0)

    def fix(i, _):
        sl = pl.ds(_mo(i * LANES, LANES), LANES)
        v = agg_tab[sl]
        agg_tab[sl] = jnp.where(v <= jnp.float32(-3.0e38), 0.0, v)
        return 0
    lax.fori_loop(0, RPT * (D // LANES), fix, 0)

    pltpu.sync_copy(agg_tab, agg_out.at[pl.ds(_mo(rlo * D), RPT * D)])


def _run_k5b(msgf, rowS, off):
    kern = pl.kernel(
        _k5b_agg,
        out_type=_f32((NPAD * D,)),
        mesh=_mesh,
        compiler_params=_SC_PARAMS,
        scratch_types=[
            pltpu.VMEM((RPT * D,), jnp.float32),
            pltpu.VMEM((W5,), jnp.int32),
            pltpu.VMEM((W5 * D,), jnp.float32),
            pltpu.VMEM((LANES,), jnp.int32),
        ],
        name="k5b_aggmax",
    )
    return kern(msgf, rowS, off)


def _k3b_gather(rowS_hbm, colS_hbm, ms_hbm, v_hbm,
                msr_out, vc_out,
                rowf_v, colf_v, rowj_v, colj_v, mswin, vwin, sem):
    wid = _wid()
    eb = _mo(wid * EW)
    nch = EW // GW
    nv = GW // LANES

    def load_clamp(src_hbm, dst, hi):
        pltpu.sync_copy(src_hbm.at[pl.ds(eb, EW)], dst)

        def body(i, _):
            sl = pl.ds(_mo(i * LANES, LANES), LANES)
            dst[sl] = jnp.clip(dst[sl], 0, hi)
            return 0
        lax.fori_loop(0, EW // LANES, body, 0)

    load_clamp(rowS_hbm, rowf_v, NPAD - 1)
    load_clamp(colS_hbm, colf_v, N - 1)

    def win(j, _):
        cb = _mo(j * GW)

        def cp(i2, _):
            sls = pl.ds(cb + _mo(i2 * LANES, LANES), LANES)
            sld = pl.ds(_mo(i2 * LANES, LANES), LANES)
            rowj_v[sld] = rowf_v[sls]
            colj_v[sld] = colf_v[sls]
            return 0
        lax.fori_loop(0, nv, cp, 0)

        pltpu.async_copy(ms_hbm.at[rowj_v], mswin, sem).wait()
        pltpu.sync_copy(mswin, msr_out.at[pl.ds(eb + cb, GW)])
        pltpu.async_copy(v_hbm.at[colj_v], vwin, sem).wait()
        pltpu.sync_copy(vwin, vc_out.at[pl.ds(eb + cb, GW)])
        return 0
    lax.fori_loop(0, nch, win, 0)


def _run_k3b(rowS, colS, ms_tab, v_tab):
    kern = pl.kernel(
        _k3b_gather,
        out_type=(_f32((EP2, D)), _f32((EP2, D))),
        mesh=_mesh,
        compiler_params=_SC_PARAMS,
        scratch_types=[
            pltpu.VMEM((EW,), jnp.int32),
            pltpu.VMEM((EW,), jnp.int32),
            pltpu.VMEM((GW,), jnp.int32),
            pltpu.VMEM((GW,), jnp.int32),
            pltpu.VMEM((GW, D), jnp.float32),
            pltpu.VMEM((GW, D), jnp.float32),
            pltpu.SemaphoreType.DMA,
        ],
        name="k3b_gather",
    )
    return kern(rowS, colS, ms_tab, v_tab)


def _k3c_unperm(inv_hbm, ue_hbm, out_hbm, invf_v, invj_v, uwin, sem):
    wid = _wid()
    eb = _mo(wid * EW)
    nch = EW // GW
    nv = GW // LANES

    pltpu.sync_copy(inv_hbm.at[pl.ds(eb, EW)], invf_v)

    def body(i, _):
        sl = pl.ds(_mo(i * LANES, LANES), LANES)
        invf_v[sl] = jnp.clip(invf_v[sl], 0, E - 1)
        return 0
    lax.fori_loop(0, EW // LANES, body, 0)

    def win(j, _):
        cb = _mo(j * GW)

        def cp(i2, _):
            sls = pl.ds(cb + _mo(i2 * LANES, LANES), LANES)
            sld = pl.ds(_mo(i2 * LANES, LANES), LANES)
            invj_v[sld] = invf_v[sls]
            return 0
        lax.fori_loop(0, nv, cp, 0)
        pltpu.async_copy(ue_hbm.at[invj_v], uwin, sem).wait()
        pltpu.sync_copy(uwin, out_hbm.at[pl.ds(eb + cb, GW)])
        return 0
    lax.fori_loop(0, nch, win, 0)


def _run_k3c(inv_p, ue_s):
    kern = pl.kernel(
        _k3c_unperm,
        out_type=_f32((EP2, D)),
        mesh=_mesh,
        compiler_params=_SC_PARAMS,
        scratch_types=[
            pltpu.VMEM((EW,), jnp.int32),
            pltpu.VMEM((GW,), jnp.int32),
            pltpu.VMEM((GW, D), jnp.float32),
            pltpu.SemaphoreType.DMA,
        ],
        name="k3c_unpermute",
    )
    return kern(inv_p, ue_s)


W6 = 80


def _k6_sums(ue_hbm, row_hbm, col_hbm, sum_out, sum_in, cnt_in,
             uew, riw, ones_w, zb, zc, tab_sh, cnt_sh, sem):
    c = lax.axis_index("c")
    s = lax.axis_index("s")
    eb = _mo(s * ECH)
    rows_per_tile = NPAD // NSUB

    nvz = D // LANES

    def zb_fill(i, _):
        zb[i // nvz, pl.ds(_mo((i % nvz) * LANES, LANES), LANES)] = (
            jnp.zeros((LANES,), jnp.float32))
        return 0
    lax.fori_loop(0, 64 * nvz, zb_fill, 0)

    def z(i, _):
        pltpu.sync_copy(
            zb, tab_sh.at[pl.ds(_mo(s * rows_per_tile + i * 64), 64)])
        return 0
    lax.fori_loop(0, rows_per_tile // 64, z, 0)

    def zc_fill(i, _):
        zc[pl.ds(_mo(i * LANES, LANES), LANES)] = jnp.zeros(
            (LANES,), jnp.float32)
        return 0
    lax.fori_loop(0, rows_per_tile // LANES, zc_fill, 0)
    pltpu.sync_copy(zc, cnt_sh.at[pl.ds(_mo(s * rows_per_tile),
                                        rows_per_tile)])

    def ones_fill(i, _):
        ones_w[pl.ds(_mo(i * LANES, LANES), LANES)] = jnp.ones(
            (LANES,), jnp.float32)
        return 0
    lax.fori_loop(0, W6 // LANES, ones_fill, 0)

    plsc.subcore_barrier()

    def win(w, _):
        base = _mo(eb + w * W6)
        pltpu.sync_copy(ue_hbm.at[pl.ds(base, W6)], uew)

        @pl.when(c == 0)
        def _():
            pltpu.sync_copy(row_hbm.at[pl.ds(base, W6)], riw)

        @pl.when(c == 1)
        def _():
            pltpu.sync_copy(col_hbm.at[pl.ds(base, W6)], riw)

        pltpu.sync_copy(uew, tab_sh.at[riw], add=True)
        pltpu.sync_copy(ones_w, cnt_sh.at[riw], add=True)
        return 0
    lax.fori_loop(0, ECH // W6, win, 0)

    plsc.subcore_barrier()

    sl = pl.ds(_mo(s * rows_per_tile), rows_per_tile)
    slc = sl

    @pl.when(c == 0)
    def _():
        pltpu.sync_copy(tab_sh.at[sl], sum_out.at[sl])

    @pl.when(c == 1)
    def _():
        pltpu.sync_copy(tab_sh.at[sl], sum_in.at[sl])
        pltpu.sync_copy(cnt_sh.at[slc], cnt_in.at[slc])


def _run_k6(ue, row, col):
    kern = pl.kernel(
        _k6_sums,
        out_type=(_f32((NPAD, D)), _f32((NPAD, D)), _f32((NPAD,))),
        mesh=_mesh,
        compiler_params=_SC_PARAMS,
        scratch_types=[
            pltpu.VMEM((W6, D), jnp.float32),
            pltpu.VMEM((W6,), jnp.int32),
            pltpu.VMEM((W6,), jnp.float32),
            pltpu.VMEM((64, D), jnp.float32),
            pltpu.VMEM((NPAD // NSUB,), jnp.float32),
            pltpu.VMEM_SHARED((NPAD, D), jnp.float32),
            pltpu.VMEM_SHARED((NPAD,), jnp.float32),
            pltpu.SemaphoreType.DMA,
        ],
        name="k6_twin_sums",
    )
    return kern(ue, row, col)


def _k0a_pt(cn_ref, cr_ref, wt, bt, out):
    cnf = cn_ref[...]
    crf = cr_ref[...]
    a = jnp.repeat(cnf, 160, axis=0)
    b = jnp.tile(jnp.repeat(crf, 20, axis=0), (20, 1))
    cpart = jnp.tile(cnf, (160, 1))
    te = a + b + cpart
    nrm = jnp.sqrt(jnp.sum(te * te, axis=1, keepdims=True))
    te = te / (nrm + 1e-8)
    out[...] = te @ wt[...] + bt[...]


def _run_k0a(clip_node, clip_rel, wt, bt):
    return pl.pallas_call(
        _k0a_pt,
        out_shape=_f32((NCOMBO, D)),
    )(clip_node, clip_rel, wt, bt.reshape(1, D))


def _k0b_tabs(x_blk, pos_blk, wq, bq, wv, bv, row_tab, col_tab, v_tab):
    x = x_blk[...]
    p16 = pos_blk[...]
    q = x @ wq[...] + bq[...]
    v = x @ wv[...] + bv[...]
    zr = jnp.zeros((x.shape[0], TA - 2 * D - 16), jnp.float32)
    row_tab[...] = jnp.concatenate([q, x, p16, zr], axis=1)
    col_tab[...] = jnp.concatenate([x, v, p16, zr], axis=1)
    v_tab[...] = v


def _run_k0b(x, pos16, wq, bq, wv, bv):
    nb = N // 1000
    return pl.pallas_call(
        _k0b_tabs,
        grid=(nb,),
        in_specs=[
            pl.BlockSpec((1000, D), lambda i: (i, 0)),
            pl.BlockSpec((1000, 16), lambda i: (i, 0)),
            pl.BlockSpec((D, D), lambda i: (0, 0)),
            pl.BlockSpec((1, D), lambda i: (0, 0)),
            pl.BlockSpec((D, D), lambda i: (0, 0)),
            pl.BlockSpec((1, D), lambda i: (0, 0)),
        ],
        out_specs=[
            pl.BlockSpec((1000, TA), lambda i: (i, 0)),
            pl.BlockSpec((1000, TB), lambda i: (i, 0)),
            pl.BlockSpec((1000, D), lambda i: (i, 0)),
        ],
        out_shape=[_f32((N, TA)), _f32((N, TB)), _f32((N, D))],
    )(x, pos16, wq, bq.reshape(1, D), wv, bv.reshape(1, D))


BE = 640


def _k4_edge(a_ref, b_ref, t_ref, r_ref, mf_ref, ef_ref,
             wk, bk, wqk, wkk, wtk, b1k, w2s, b2s,
             dw1a, dw1b, db1, dw2, db2,
             w1a, w1b, w1c, w1d, eub1, euw2, eub2,
             lp_out, ue_out):
    a = a_ref[...]
    b = b_ref[...]
    q = a[:, 0:D]
    xr = a[:, D:2 * D]
    pr = a[:, 2 * D:2 * D + 16]
    xc = b[:, 0:D]
    pc = b[:, 2 * D:2 * D + 16]
    ef = ef_ref[...]
    t = t_ref[...]

    k = ef @ wk[...] + bk[...]
    h1 = jnp.maximum(
        q @ wqk[...] + k @ wkk[...] + t @ wtk[...] + b1k[...], 0.0)
    lg = h1 @ w2s[...] + b2s[...]

    diff = pr - pc
    dist = jnp.sqrt(jnp.sum(diff * diff, axis=1, keepdims=True) + 1e-12)
    hd = jnp.maximum(diff @ dw1a[...] + dist * dw1b[...] + db1[...], 0.0)
    dm = jax.nn.sigmoid(hd @ dw2[...] + db2[...])

    lp_out[...] = jnp.concatenate(
        [lg, dm, jnp.zeros((lg.shape[0], 15), jnp.float32)], axis=1)

    rev = r_ref[...] * mf_ref[...]
    hu = jnp.maximum(
        xr @ w1a[...] + xc @ w1b[...] + ef @ w1c[...] + rev @ w1d[...]
        + eub1[...], 0.0)
    ue_out[...] = hu @ euw2[...] + eub2[...]


def _run_k4(a, b, t, r, mf, efs, wk, bk, wqk, wkk, wtk, b1k, w2s, b2s,
            dw1a, dw1b, db1, dw2, db2, w1a, w1b, w1c, w1d, eub1, euw2, eub2):
    nb = E // BE
    full = lambda shape: pl.BlockSpec(shape, lambda i: (0, 0))
    return pl.pallas_call(
        _k4_edge,
        grid=(nb,),
        in_specs=[
            pl.BlockSpec((BE, TA), lambda i: (i, 0)),
            pl.BlockSpec((BE, TB), lambda i: (i, 0)),
            pl.BlockSpec((BE, D), lambda i: (i, 0)),
            pl.BlockSpec((BE, D), lambda i: (i, 0)),
            pl.BlockSpec((BE, 1), lambda i: (i, 0)),
            pl.BlockSpec((BE, D), lambda i: (i, 0)),
            full((D, D)), full((1, D)),
            full((D, 640)), full((D, 640)), full((D, 640)),
            full((1, 640)), full((640, 16)), full((1, 16)),
            full((16, 32)), full((1, 32)), full((1, 32)),
            full((32, 1)), full((1, 1)),
            full((D, 384)), full((D, 384)), full((D, 384)), full((D, 384)),
            full((1, 384)), full((384, D)), full((1, D)),
        ],
        out_specs=[
            pl.BlockSpec((BE, 32), lambda i: (i, 0)),
            pl.BlockSpec((BE, D), lambda i: (i, 0)),
        ],
        out_shape=[_f32((E, 32)), _f32((E, D))],
    )(a, b, t, r, mf, efs, wk, bk, wqk, wkk, wtk, b1k, w2s, b2s,
      dw1a, dw1b, db1, dw2, db2, w1a, w1b, w1c, w1d, eub1, euw2, eub2)


def _k4b_msg(lp_ref, msr_ref, vc_ref, selm, sels, prc, msg_out):
    lp = lp_ref[...]
    lg = lp[:, 0:16]
    dm = lp[:, 16:17]
    msr = msr_ref[...]
    m = msr @ selm[...]
    s = msr @ sels[...]
    p = jnp.exp(lg - m) / (s + 1e-9)
    alpha = (p @ prc[...]) * dm
    msg_out[...] = vc_ref[...] * alpha


def _run_k4b(lp, msr, vc, selm, sels, prc):
    nb = E // BE
    full = lambda shape: pl.BlockSpec(shape, lambda i: (0, 0))
    return pl.pallas_call(
        _k4b_msg,
        grid=(nb,),
        in_specs=[
            pl.BlockSpec((BE, 32), lambda i: (i, 0)),
            pl.BlockSpec((BE, D), lambda i: (i, 0)),
            pl.BlockSpec((BE, D), lambda i: (i, 0)),
            full((D, 16)), full((D, 16)), full((16, D)),
        ],
        out_specs=pl.BlockSpec((BE, D), lambda i: (i, 0)),
        out_shape=_f32((E, D)),
    )(lp, msr, vc, selm, sels, prc)


def _k7_node(x_ref, agg_ref, so_ref, si_ref, co_ref, ci_ref,
             nw1a, nw1b, nb1, nw2, nb2, eawa, eawb, eab, out):
    x = x_ref[...]
    agg = agg_ref[...]
    h = jnp.maximum(x @ nw1a[...] + agg @ nw1b[...] + nb1[...], 0.0)
    un = h @ nw2[...] + nb2[...]
    co = jnp.maximum(co_ref[...], 1.0)
    ci = jnp.maximum(ci_ref[...], 1.0)
    om = so_ref[...] / co
    im = si_ref[...] / ci
    gate = jax.nn.sigmoid(om @ eawa[...] + im @ eawb[...] + eab[...])
    out[...] = un * gate


def _run_k7(x, agg, so, si, co, ci, nw1a, nw1b, nb1, nw2, nb2,
            eawa, eawb, eab):
    nb = N // 1000
    full = lambda shape: pl.BlockSpec(shape, lambda i: (0, 0))
    return pl.pallas_call(
        _k7_node,
        grid=(nb,),
        in_specs=[
            pl.BlockSpec((1000, D), lambda i: (i, 0)),
            pl.BlockSpec((1000, D), lambda i: (i, 0)),
            pl.BlockSpec((1000, D), lambda i: (i, 0)),
            pl.BlockSpec((1000, D), lambda i: (i, 0)),
            pl.BlockSpec((1000, 1), lambda i: (i, 0)),
            pl.BlockSpec((1000, 1), lambda i: (i, 0)),
            full((D, 256)), full((D, 256)), full((1, 256)),
            full((256, D)), full((1, D)),
            full((D, D)), full((D, D)), full((1, D)),
        ],
        out_specs=pl.BlockSpec((1000, D), lambda i: (i, 0)),
        out_shape=_f32((N, D)),
    )(x, agg, so, si, co, ci, nw1a, nw1b, nb1, nw2, nb2, eawa, eawb, eab)


def kernel(x, edge_feature, node_positions, params, edge_index,
           gt_rel_label, gt_obj_label):
    p = params
    row = edge_index[0].astype(jnp.int32)
    col = edge_index[1].astype(jnp.int32)
    obj = gt_obj_label.astype(jnp.int32)
    rel = gt_rel_label.astype(jnp.int32)
    pos16 = jnp.pad(node_positions, ((0, 0), (0, 13)))

    eye = jnp.eye(H, dtype=jnp.float32)
    wqk = jnp.concatenate([jnp.kron(eye, p['a3W1'][:DH]),
                           jnp.kron(eye, p['atW1'][:DH])], axis=1)
    wkk = jnp.concatenate([jnp.kron(eye, p['a3W1'][DH:2 * DH]),
                           jnp.kron(eye, p['atW1'][DH:2 * DH])], axis=1)
    wtk = jnp.concatenate([jnp.zeros((D, 256), jnp.float32),
                           jnp.kron(eye, p['atW1'][2 * DH:3 * DH])], axis=1)
    b1k = jnp.concatenate([jnp.tile(p['a3b1'], H),
                           jnp.tile(p['atb1'], H)]).reshape(1, 640)
    w2s_a = jnp.kron(eye, p['a3W2'].sum(axis=1)[:, None])
    w2s_t = jnp.kron(eye, p['atW2'].sum(axis=1)[:, None])
    w2s = jnp.concatenate([
        jnp.concatenate([w2s_a, jnp.zeros((256, 8), jnp.float32)], axis=1),
        jnp.concatenate([jnp.zeros((384, 8), jnp.float32), w2s_t], axis=1),
    ], axis=0) / TEMP
    b2s = jnp.concatenate([
        jnp.full((8,), p['a3b2'].sum(), jnp.float32),
        jnp.full((8,), p['atb2'].sum(), jnp.float32)]).reshape(1, 16) / TEMP

    dw1a = jnp.pad(p['dW1'][:3], ((0, 13), (0, 0)))
    dw1b = p['dW1'][3].reshape(1, 32)
    db1 = p['db1'].reshape(1, 32)
    dw2 = p['dW2']
    db2 = p['db2'].reshape(1, 1)

    w1a = p['euW1'][0:D]
    w1b = p['euW1'][D:2 * D]
    w1c = p['euW1'][2 * D:3 * D]
    w1d = p['euW1'][3 * D:4 * D]
    eub1 = p['eub1'].reshape(1, 384)
    eub2 = p['eub2'].reshape(1, D)

    nw1a = p['nuW1'][0:D]
    nw1b = p['nuW1'][D:2 * D]
    nb1 = p['nub1'].reshape(1, 256)
    nb2 = p['nub2'].reshape(1, D)
    eawa = p['eaW'][0:D]
    eawb = p['eaW'][D:2 * D]
    eab = p['eab'].reshape(1, D)

    selm = jnp.zeros((D, 16), jnp.float32).at[:16, :].set(jnp.eye(16))
    sels = jnp.zeros((D, 16), jnp.float32).at[16:32, :].set(jnp.eye(16))
    prc = jnp.concatenate([jnp.kron(eye, jnp.ones((1, DH), jnp.float32)),
                           jnp.kron(eye, jnp.ones((1, DH), jnp.float32))],
                          axis=0) * 0.5

    pt_tab = _run_k0a(p['clip_node'], p['clip_rel'], p['Wt'], p['bt'])
    row_tab, col_tab, v_tab = _run_k0b(x, pos16, p['Wq'], p['bq'],
                                       p['Wv'], p['bv'])
    rowS, colS, idxS, cnt, off, inv2d = _run_k1(row, col)
    ridxS, matchS = _run_k2(rowS, colS, off, cnt, idxS)
    a_g, b_g, t_g, r_g, efs = _run_k3(rowS, colS, idxS, obj, rel, ridxS,
                                      row_tab, col_tab, pt_tab, edge_feature)

    mf = matchS.astype(jnp.float32).reshape(EP2, 1)
    lp, ue_s = _run_k4(a_g, b_g, t_g, r_g, mf, efs,
                       p['Wk'], p['bk'].reshape(1, D), wqk, wkk, wtk, b1k,
                       w2s, b2s, dw1a, dw1b, db1, dw2, db2,
                       w1a, w1b, w1c, w1d, eub1, p['euW2'], eub2)

    ms_tab = _run_k5a(lp.reshape(E * 32), rowS, off).reshape(NPAD, D)
    msr, vc = _run_k3b(rowS, colS, ms_tab, v_tab)
    msg = _run_k4b(lp, msr, vc, selm, sels, prc)
    agg = _run_k5b(msg.reshape(E * D), rowS, off).reshape(NPAD, D)

    inv_p = jnp.pad(inv2d[:E], (0, EP2 - E))
    ue = _run_k3c(inv_p, ue_s)[:E]

    sum_out, sum_in, cnt_in = _run_k6(ue, row, col)

    updated_node = _run_k7(
        x, agg[:N], sum_out[:N], sum_in[:N],
        cnt[:N].astype(jnp.float32).reshape(N, 1), cnt_in[:N].reshape(N, 1),
        nw1a, nw1b, nb1, p['nuW2'], nb2, eawa, eawb, eab)

    return updated_node, ue


# speedup vs baseline: 1.0080x; 1.0080x over previous
import jax
import jax.numpy as jnp
from jax import lax
from jax.experimental import pallas as pl
from jax.experimental.pallas import tpu as pltpu
from jax.experimental.pallas import tpu_sc as plsc

N = 10000
E = 160000
D = 128
H = 8
DH = 16
TEMP = 4.0

NCORE = 2
NSUB = 16
NW = NCORE * NSUB
LANES = 16

RPT = 320
NPAD = NW * RPT
RPC = NPAD // 2
ECH = E // NSUB
EW = 5120
EP2 = NW * EW
IDXW = 80
GW = 128
GW3 = 64
TA = 384
TB = 384
NCOMBO = 3200
W5 = 512

_mesh = plsc.VectorSubcoreMesh(core_axis_name="c", subcore_axis_name="s")
_SC_PARAMS = pltpu.CompilerParams(needs_layout_passes=False)


def _f32(shape):
    return jax.ShapeDtypeStruct(shape, jnp.float32)


def _i32(shape):
    return jax.ShapeDtypeStruct(shape, jnp.int32)


def _wid():
    return lax.axis_index("s") * NCORE + lax.axis_index("c")


def _mo(x, n=8):
    return pl.multiple_of(x, n)


def _zero_i32(ref, n):
    def body(i, _):
        ref[pl.ds(i * LANES, LANES)] = jnp.zeros((LANES,), jnp.int32)
        return 0
    lax.fori_loop(0, n // LANES, body, 0)


def _fill2d_f32(ref, rows, width, val):
    nv = width // LANES

    def body(i, _):
        ref[i // nv, pl.ds((i % nv) * LANES, LANES)] = jnp.full(
            (LANES,), val, jnp.float32)
        return 0
    lax.fori_loop(0, rows * nv, body, 0)


def _clamp_to_2d(src_flat, dst2d, total, width, lo, hi):
    nv = width // LANES

    def body(i, _):
        v = src_flat[pl.ds(i * LANES, LANES)]
        dst2d[i // nv, pl.ds((i % nv) * LANES, LANES)] = jnp.clip(v, lo, hi)
        return 0
    lax.fori_loop(0, total // LANES, body, 0)


def _k1_sort(row_hbm, col_hbm, rowS, colS, idxS, cnt_hbm, off_hbm, inv_hbm,
             rows_v, payl_v, rank_v, hist_v, tmp_v, base_v, tot_v,
             off_v, posj_v, rowj_v, colj_v, idxj_v, grid_sh):
    c = lax.axis_index("c")
    s = lax.axis_index("s")
    rlo = c * RPC
    eb = _mo(s * ECH)

    pltpu.sync_copy(row_hbm.at[pl.ds(eb, ECH)], rows_v)
    _zero_i32(hist_v, RPC)

    def hist_body(i, _):
        sl = pl.ds(_mo(i * LANES, LANES), LANES)
        r = rows_v[sl]
        m = (r >= rlo) & (r < rlo + RPC)
        rl = jnp.where(m, r - rlo, 0)
        occ, last = plsc.scan_count(rl, mask=m)
        base = plsc.load_gather(hist_v, [rl])
        rank_v[sl] = base + occ - 1
        plsc.store_scatter(hist_v, [rl], base + occ, mask=last & m)
        return 0
    lax.fori_loop(0, ECH // LANES, hist_body, 0)

    pltpu.sync_copy(hist_v, grid_sh.at[pl.ds(_mo(s * RPC), RPC)])
    plsc.subcore_barrier()

    _zero_i32(base_v, RPC)
    _zero_i32(tot_v, RPC)
    for sp in range(NSUB):
        pltpu.sync_copy(grid_sh.at[pl.ds(sp * RPC, RPC)], tmp_v)
        use = jnp.int32(sp) < s

        def acc_body(i, _):
            sl = pl.ds(_mo(i * LANES, LANES), LANES)
            tv = tmp_v[sl]
            base_v[sl] = base_v[sl] + jnp.where(use, tv, 0)
            tot_v[sl] = tot_v[sl] + tv
            return 0
        lax.fori_loop(0, RPC // LANES, acc_body, 0)

    def sum_body(i, acc):
        return acc + jnp.sum(tot_v[pl.ds(_mo(i * LANES, LANES), LANES)])
    core_total = lax.fori_loop(0, RPC // LANES, sum_body, jnp.int32(0))
    core_base = jnp.where(c == 0, jnp.int32(0), jnp.int32(E) - core_total)

    def scan_body(i, carry):
        sl = pl.ds(_mo(i * LANES, LANES), LANES)
        v = tot_v[sl]
        cs = plsc.cumsum(v)
        off_v[sl] = carry + core_base + cs - v
        return carry + jnp.sum(v)
    lax.fori_loop(0, RPC // LANES, scan_body, jnp.int32(0))

    sl = pl.ds(_mo(s * RPT), RPT)
    pltpu.sync_copy(tot_v.at[sl], cnt_hbm.at[pl.ds(_mo(c * RPC + s * RPT),
                                                   RPT)])
    pltpu.sync_copy(off_v.at[sl], off_hbm.at[pl.ds(_mo(c * RPC + s * RPT),
                                                   RPT)])

    pltpu.sync_copy(col_hbm.at[pl.ds(eb, ECH)], payl_v)

    def chunk(j, _):
        cb = _mo(j * IDXW)

        def vec(k2, _):
            sl_src = pl.ds(cb + _mo(k2 * LANES, LANES), LANES)
            sl_dst = pl.ds(_mo(k2 * LANES, LANES), LANES)
            r = rows_v[sl_src]
            m = (r >= rlo) & (r < rlo + RPC)
            rl = jnp.where(m, r - rlo, 0)
            pos = (plsc.load_gather(off_v, [rl])
                   + plsc.load_gather(base_v, [rl])
                   + rank_v[sl_src])
            pos = jnp.where(m, pos,
                            jnp.int32(E) + lax.iota(jnp.int32, LANES))
            posj_v[sl_dst] = pos
            rowj_v[sl_dst] = r
            colj_v[sl_dst] = payl_v[sl_src]
            idxj_v[sl_dst] = jnp.where(
                m, eb + cb + k2 * LANES + lax.iota(jnp.int32, LANES),
                jnp.int32(E) + lax.iota(jnp.int32, LANES))
            return 0
        lax.fori_loop(0, IDXW // LANES, vec, 0)
        pltpu.sync_copy(rowj_v, rowS.at[posj_v])
        pltpu.sync_copy(colj_v, colS.at[posj_v])
        pltpu.sync_copy(idxj_v, idxS.at[posj_v])
        pltpu.sync_copy(posj_v, inv_hbm.at[idxj_v])
        return 0
    lax.fori_loop(0, ECH // IDXW, chunk, 0)


def _run_k1(row, col):
    kern = pl.kernel(
        _k1_sort,
        out_type=(_i32((EP2,)), _i32((EP2,)), _i32((EP2,)),
                  _i32((NPAD,)), _i32((NPAD,)), _i32((E + LANES,))),
        mesh=_mesh,
        compiler_params=_SC_PARAMS,
        scratch_types=[
            pltpu.VMEM((ECH,), jnp.int32),
            pltpu.VMEM((ECH,), jnp.int32),
            pltpu.VMEM((ECH,), jnp.int32),
            pltpu.VMEM((RPC,), jnp.int32),
            pltpu.VMEM((RPC,), jnp.int32),
            pltpu.VMEM((RPC,), jnp.int32),
            pltpu.VMEM((RPC,), jnp.int32),
            pltpu.VMEM((RPC,), jnp.int32),
            pltpu.VMEM((IDXW,), jnp.int32),
            pltpu.VMEM((IDXW,), jnp.int32),
            pltpu.VMEM((IDXW,), jnp.int32),
            pltpu.VMEM((IDXW,), jnp.int32),
            pltpu.VMEM_SHARED((NSUB * RPC,), jnp.int32),
        ],
        name="k1_sort",
    )
    return kern(row, col)


def _k2_walk(rowS_hbm, colS_hbm, off_hbm, cnt_hbm, idxS,
             ridx_hbm, match_hbm,
             tgt_v, ptr_v, end_v, hitp_v, vals_v, res_v, flj_v, flj2_v,
             sem, sem2):
    wid = _wid()
    eb = _mo(wid * EW)
    nch = EW // GW

    pltpu.sync_copy(rowS_hbm.at[pl.ds(eb, EW)], tgt_v)
    pltpu.sync_copy(colS_hbm.at[pl.ds(eb, EW)], vals_v)

    def g_off(j, _):
        cb = _mo(j * GW)

        def cl(i2, _):
            v = vals_v[pl.ds(cb + _mo(i2 * LANES, LANES), LANES)]
            flj_v[pl.ds(_mo(i2 * LANES, LANES), LANES)] = jnp.clip(
                v, 0, NPAD - 1)
            return 0
        lax.fori_loop(0, GW // LANES, cl, 0)
        pltpu.async_copy(off_hbm.at[flj_v], ptr_v.at[pl.ds(cb, GW)],
                         sem).wait()
        pltpu.async_copy(cnt_hbm.at[flj_v], end_v.at[pl.ds(cb, GW)],
                         sem).wait()
        return 0
    lax.fori_loop(0, nch, g_off, 0)

    def init_body(i, _):
        sl = pl.ds(_mo(i * LANES, LANES), LANES)
        end_v[sl] = ptr_v[sl] + end_v[sl]
        hitp_v[sl] = jnp.full((LANES,), -1, jnp.int32)
        return 0
    lax.fori_loop(0, EW // LANES, init_body, 0)

    def walk_cond(carry):
        return carry > 0

    def walk_body(carry):
        def fill(j, dst):
            for i2 in range(GW // LANES):
                p = ptr_v[pl.ds(j * GW + i2 * LANES, LANES)]
                ok = (p >= 0) & (p < E)
                dst[pl.ds(i2 * LANES, LANES)] = jnp.where(ok, p, E)

        def update(j, n_act):
            for i2 in range(GW // LANES):
                sl = pl.ds(j * GW + i2 * LANES, LANES)
                p = ptr_v[sl]
                active = (p < end_v[sl]) & (hitp_v[sl] < 0)
                hit = active & (vals_v[sl] == tgt_v[sl])
                hitp_v[sl] = jnp.where(hit, p, hitp_v[sl])
                still = active & ~hit
                ptr_v[sl] = jnp.where(still, p + 1, p)
                n_act = n_act + jnp.sum(jnp.where(still, 1, 0))
            return n_act

        fls = [flj_v, flj2_v]
        sems = [sem, sem2]
        handles = [None, None]
        n_act = jnp.int32(0)
        for j in range(nch):
            fill(j, fls[j % 2])
            handles[j % 2] = pltpu.async_copy(
                colS_hbm.at[fls[j % 2]],
                vals_v.at[pl.ds(j * GW, GW)], sems[j % 2])
            if j > 0:
                handles[(j - 1) % 2].wait()
                n_act = update(j - 1, n_act)
        handles[(nch - 1) % 2].wait()
        n_act = update(nch - 1, n_act)
        return n_act

    lax.while_loop(walk_cond, walk_body, jnp.int32(1))

    def g_res(j, _):
        cb = _mo(j * GW)

        def cl(i2, _):
            p = hitp_v[pl.ds(cb + _mo(i2 * LANES, LANES), LANES)]
            flj_v[pl.ds(_mo(i2 * LANES, LANES), LANES)] = jnp.where(
                p >= 0, p, E)
            return 0
        lax.fori_loop(0, GW // LANES, cl, 0)
        pltpu.async_copy(idxS.at[flj_v], vals_v.at[pl.ds(cb, GW)],
                         sem).wait()
        return 0
    lax.fori_loop(0, nch, g_res, 0)

    def fin(i, _):
        sl = pl.ds(_mo(i * LANES, LANES), LANES)
        ok = hitp_v[sl] >= 0
        res_v[sl] = jnp.clip(jnp.where(ok, vals_v[sl], 0), 0, E - 1)
        ptr_v[sl] = jnp.where(ok, 1, 0)
        return 0
    lax.fori_loop(0, EW // LANES, fin, 0)

    pltpu.sync_copy(res_v, ridx_hbm.at[pl.ds(eb, EW)])
    pltpu.sync_copy(ptr_v, match_hbm.at[pl.ds(eb, EW)])


def _run_k2(rowS, colS, off, cnt, idxS):
    kern = pl.kernel(
        _k2_walk,
        out_type=(_i32((EP2,)), _i32((EP2,))),
        mesh=_mesh,
        compiler_params=_SC_PARAMS,
        scratch_types=[
            pltpu.VMEM((EW,), jnp.int32),
            pltpu.VMEM((EW,), jnp.int32),
            pltpu.VMEM((EW,), jnp.int32),
            pltpu.VMEM((EW,), jnp.int32),
            pltpu.VMEM((EW,), jnp.int32),
            pltpu.VMEM((EW,), jnp.int32),
            pltpu.VMEM((GW,), jnp.int32),
            pltpu.VMEM((GW,), jnp.int32),
            pltpu.SemaphoreType.DMA,
            pltpu.SemaphoreType.DMA,
        ],
        name="k2_revlookup",
    )
    return kern(rowS, colS, off, cnt, idxS)


def _k3_gather(rowS_hbm, colS_hbm, idxS_hbm, obj_hbm, rel_hbm, ridx_hbm,
               row_tab, col_tab, pt_tab, ef_hbm,
               a_out, b_out, t_out, r_out, efs_out,
               rowf_v, colf_v, idxf_v, ridxf_v,
               rowj_v, colj_v, idxj_v, ridxj_v, comboj_v, objcj_v, relj_v,
               awin, bwin, twin, rwin, ewin, sem, sem2, sem3, sem4, sem5):
    wid = _wid()
    eb = _mo(wid * EW)
    nch = EW // GW3
    nv = GW3 // LANES

    def load_clamp(src_hbm, dst, hi):
        pltpu.sync_copy(src_hbm.at[pl.ds(eb, EW)], dst)

        def body(i, _):
            sl = pl.ds(_mo(i * LANES, LANES), LANES)
            dst[sl] = jnp.clip(dst[sl], 0, hi)
            return 0
        lax.fori_loop(0, EW // LANES, body, 0)

    load_clamp(rowS_hbm, rowf_v, N - 1)
    load_clamp(colS_hbm, colf_v, N - 1)
    load_clamp(idxS_hbm, idxf_v, E - 1)
    load_clamp(ridx_hbm, ridxf_v, E - 1)

    def chunk(j, _):
        cb = _mo(j * GW3)

        def cp(i2, _):
            sls = pl.ds(cb + _mo(i2 * LANES, LANES), LANES)
            sld = pl.ds(_mo(i2 * LANES, LANES), LANES)
            rowj_v[sld] = rowf_v[sls]
            colj_v[sld] = colf_v[sls]
            idxj_v[sld] = idxf_v[sls]
            ridxj_v[sld] = ridxf_v[sls]
            return 0
        lax.fori_loop(0, nv, cp, 0)

        pltpu.async_copy(obj_hbm.at[rowj_v], comboj_v, sem).wait()
        pltpu.async_copy(obj_hbm.at[colj_v], objcj_v, sem).wait()
        pltpu.async_copy(rel_hbm.at[idxj_v], relj_v, sem).wait()

        def mix(i2, _):
            sl = pl.ds(_mo(i2 * LANES, LANES), LANES)
            comboj_v[sl] = (comboj_v[sl] * 160 + relj_v[sl] * 20
                            + objcj_v[sl])
            return 0
        lax.fori_loop(0, nv, mix, 0)

        ob = pl.ds(eb + cb, GW3)
        ha = pltpu.async_copy(row_tab.at[rowj_v], awin, sem)
        hb = pltpu.async_copy(col_tab.at[colj_v], bwin, sem2)
        ht = pltpu.async_copy(pt_tab.at[comboj_v], twin, sem3)
        he = pltpu.async_copy(ef_hbm.at[idxj_v], ewin, sem4)
        hr = pltpu.async_copy(ef_hbm.at[ridxj_v], rwin, sem5)
        ha.wait()
        pltpu.sync_copy(awin, a_out.at[ob])
        hb.wait()
        pltpu.sync_copy(bwin, b_out.at[ob])
        ht.wait()
        pltpu.sync_copy(twin, t_out.at[ob])
        he.wait()
        pltpu.sync_copy(ewin, efs_out.at[ob])
        hr.wait()
        pltpu.sync_copy(rwin, r_out.at[ob])
        return 0
    lax.fori_loop(0, nch, chunk, 0)


def _run_k3(rowS, colS, idxS, obj, rel, ridx, row_tab, col_tab, pt_tab,
            ef):
    kern = pl.kernel(
        _k3_gather,
        out_type=(_f32((EP2, TA)), _f32((EP2, TB)), _f32((EP2, D)),
                  _f32((EP2, D)), _f32((EP2, D))),
        mesh=_mesh,
        compiler_params=_SC_PARAMS,
        scratch_types=[
            pltpu.VMEM((EW,), jnp.int32),
            pltpu.VMEM((EW,), jnp.int32),
            pltpu.VMEM((EW,), jnp.int32),
            pltpu.VMEM((EW,), jnp.int32),
            pltpu.VMEM((GW3,), jnp.int32),
            pltpu.VMEM((GW3,), jnp.int32),
            pltpu.VMEM((GW3,), jnp.int32),
            pltpu.VMEM((GW3,), jnp.int32),
            pltpu.VMEM((GW3,), jnp.int32),
            pltpu.VMEM((GW3,), jnp.int32),
            pltpu.VMEM((GW3,), jnp.int32),
            pltpu.VMEM((GW3, TA), jnp.float32),
            pltpu.VMEM((GW3, TB), jnp.float32),
            pltpu.VMEM((GW3, D), jnp.float32),
            pltpu.VMEM((GW3, D), jnp.float32),
            pltpu.VMEM((GW3, D), jnp.float32),
            pltpu.SemaphoreType.DMA,
            pltpu.SemaphoreType.DMA,
            pltpu.SemaphoreType.DMA,
            pltpu.SemaphoreType.DMA,
            pltpu.SemaphoreType.DMA,
        ],
        name="k3_gather",
    )
    return kern(rowS, colS, idxS, obj, rel, ridx, row_tab, col_tab,
                pt_tab, ef)


def _k5a_ms(lpf_hbm, rowS_hbm, off_hbm, ms_out,
            m_tab, s_tab, ms_buf, rows_w, lpw, ob):
    wid = _wid()
    rlo = wid * RPT

    pltpu.sync_copy(off_hbm.at[pl.ds(_mo(rlo), LANES)], ob)
    start = ob[pl.ds(0, LANES)][0]
    is_last = wid == NW - 1
    nxt = _mo(jnp.where(is_last, NPAD - LANES, rlo + RPT))
    pltpu.sync_copy(off_hbm.at[pl.ds(nxt, LANES)], ob)
    end = jnp.where(is_last, jnp.int32(E), ob[pl.ds(0, LANES)][0])

    abase = start - lax.rem(start, jnp.int32(8))
    nwin = (end - abase + (W5 - 1)) // W5

    def fill(ref, n, val):
        def body(i, _):
            ref[pl.ds(_mo(i * LANES, LANES), LANES)] = jnp.full(
                (LANES,), val, jnp.float32)
            return 0
        lax.fori_loop(0, n // LANES, body, 0)

    fill(m_tab, RPT * LANES, -jnp.inf)
    fill(s_tab, RPT * LANES, 0.0)

    def win_common(w):
        base_u = abase + w * W5
        base = _mo(jnp.minimum(base_u, jnp.int32(E - W5)))
        pltpu.sync_copy(lpf_hbm.at[pl.ds(_mo(base * 32), W5 * 32)], lpw)
        pltpu.sync_copy(rowS_hbm.at[pl.ds(base, W5)], rows_w)
        return base_u, base

    def win_a(w, _):
        base_u, base = win_common(w)

        def edge(i16, _):
            rvec = rows_w[pl.ds(_mo(i16 * LANES, LANES), LANES)]
            for k in range(LANES):
                i = i16 * LANES + k
                rloc = rvec[k] - rlo
                pp = base + i
                ok = ((rloc >= 0) & (rloc < RPT) & (pp >= start)
                      & (pp < end) & (pp >= base_u))
                rc = jnp.clip(rloc, 0, RPT - 1)
                lv = lpw[pl.ds(_mo(i * 32, LANES), LANES)]
                msl = pl.ds(_mo(rc * LANES, LANES), LANES)
                cur = m_tab[msl]
                m_tab[msl] = jnp.where(ok, jnp.maximum(cur, lv), cur)
            return 0
        lax.fori_loop(0, W5 // LANES, edge, 0)
        return 0
    lax.fori_loop(0, nwin, win_a, 0)

    def win_b(w, _):
        base_u, base = win_common(w)

        def edge(i16, _):
            rvec = rows_w[pl.ds(_mo(i16 * LANES, LANES), LANES)]
            for k in range(LANES):
                i = i16 * LANES + k
                rloc = rvec[k] - rlo
                pp = base + i
                ok = ((rloc >= 0) & (rloc < RPT) & (pp >= start)
                      & (pp < end) & (pp >= base_u))
                rc = jnp.clip(rloc, 0, RPT - 1)
                lv = lpw[pl.ds(_mo(i * 32, LANES), LANES)]
                msl = pl.ds(_mo(rc * LANES, LANES), LANES)
                ev = jnp.exp(lv - m_tab[msl])
                s_tab[msl] = s_tab[msl] + jnp.where(ok, ev, 0.0)
            return 0
        lax.fori_loop(0, W5 // LANES, edge, 0)
        return 0
    lax.fori_loop(0, nwin, win_b, 0)

    fill(ms_buf, RPT * D, 0.0)

    def pack(r, _):
        sl = pl.ds(_mo(r * LANES, LANES), LANES)
        ms_buf[pl.ds(_mo(r * D), LANES)] = m_tab[sl]
        ms_buf[pl.ds(_mo(r * D + LANES), LANES)] = s_tab[sl]
        return 0
    lax.fori_loop(0, RPT, pack, 0)

    pltpu.sync_copy(ms_buf, ms_out.at[pl.ds(_mo(rlo * D), RPT * D)])


def _run_k5a(lpf, rowS, off):
    kern = pl.kernel(
        _k5a_ms,
        out_type=_f32((NPAD * D,)),
        mesh=_mesh,
        compiler_params=_SC_PARAMS,
        scratch_types=[
            pltpu.VMEM((RPT * LANES,), jnp.float32),
            pltpu.VMEM((RPT * LANES,), jnp.float32),
            pltpu.VMEM((RPT * D,), jnp.float32),
            pltpu.VMEM((W5,), jnp.int32),
            pltpu.VMEM((W5 * 32,), jnp.float32),
            pltpu.VMEM((LANES,), jnp.int32),
        ],
        name="k5a_softmax_stats",
    )
    return kern(lpf, rowS, off)


NEGBIG = -3.4e38


def _k5b_agg(msgf_hbm, rowS_hbm, off_hbm, agg_out,
             agg_tab, rows_w, mw, ob):
    wid = _wid()
    rlo = wid * RPT

    pltpu.sync_copy(off_hbm.at[pl.ds(_mo(rlo), LANES)], ob)
    start = ob[pl.ds(0, LANES)][0]
    is_last = wid == NW - 1
    nxt = _mo(jnp.where(is_last, NPAD - LANES, rlo + RPT))
    pltpu.sync_copy(off_hbm.at[pl.ds(nxt, LANES)], ob)
    end = jnp.where(is_last, jnp.int32(E), ob[pl.ds(0, LANES)][0])

    abase = start - lax.rem(start, jnp.int32(8))
    nwin = (end - abase + (W5 - 1)) // W5

    def fill(ref, n, val):
        def body(i, _):
            ref[pl.ds(_mo(i * LANES, LANES), LANES)] = jnp.full(
                (LANES,), val, jnp.float32)
            return 0
        lax.fori_loop(0, n // LANES, body, 0)

    fill(agg_tab, RPT * D, NEGBIG)

    def win(w, _):
        base_u = abase + w * W5
        base = _mo(jnp.minimum(base_u, jnp.int32(E - W5)))
        pltpu.sync_copy(msgf_hbm.at[pl.ds(_mo(base * D), W5 * D)], mw)
        pltpu.sync_copy(rowS_hbm.at[pl.ds(base, W5)], rows_w)

        def edge(i16, _):
            rvec = rows_w[pl.ds(_mo(i16 * LANES, LANES), LANES)]
            for k in range(LANES):
                i = i16 * LANES + k
                rloc = rvec[k] - rlo
                pp = base + i
                ok = ((rloc >= 0) & (rloc < RPT) & (pp >= start)
                      & (pp < end) & (pp >= base_u))
                rc = jnp.clip(rloc, 0, RPT - 1)
                for h in range(D // LANES):
                    asl = pl.ds(_mo(rc * D + h * LANES, LANES), LANES)
                    cur = agg_tab[asl]
                    v = mw[pl.ds(_mo(i * D + h * LANES, LANES), LANES)]
                    agg_tab[asl] = jnp.where(ok, jnp.maximum(cur, v), cur)
            return 0
        lax.fori_loop(0, W5 // LANES, edge, 0)
        return 0
    lax.fori_loop(0, nwin, win, 0)

    def fix(i, _):
        sl = pl.ds(_mo(i * LANES, LANES), LANES)
        v = agg_tab[sl]
        agg_tab[sl] = jnp.where(v <= jnp.float32(-3.0e38), 0.0, v)
        return 0
    lax.fori_loop(0, RPT * (D // LANES), fix, 0)

    pltpu.sync_copy(agg_tab, agg_out.at[pl.ds(_mo(rlo * D), RPT * D)])


def _run_k5b(msgf, rowS, off):
    kern = pl.kernel(
        _k5b_agg,
        out_type=_f32((NPAD * D,)),
        mesh=_mesh,
        compiler_params=_SC_PARAMS,
        scratch_types=[
            pltpu.VMEM((RPT * D,), jnp.float32),
            pltpu.VMEM((W5,), jnp.int32),
            pltpu.VMEM((W5 * D,), jnp.float32),
            pltpu.VMEM((LANES,), jnp.int32),
        ],
        name="k5b_aggmax",
    )
    return kern(msgf, rowS, off)


def _k3b_gather(rowS_hbm, colS_hbm, ms_hbm, v_hbm,
                msr_out, vc_out,
                rowf_v, colf_v, rowj_v, colj_v, mswin, vwin, sem):
    wid = _wid()
    eb = _mo(wid * EW)
    nch = EW // GW
    nv = GW // LANES

    def load_clamp(src_hbm, dst, hi):
        pltpu.sync_copy(src_hbm.at[pl.ds(eb, EW)], dst)

        def body(i, _):
            sl = pl.ds(_mo(i * LANES, LANES), LANES)
            dst[sl] = jnp.clip(dst[sl], 0, hi)
            return 0
        lax.fori_loop(0, EW // LANES, body, 0)

    load_clamp(rowS_hbm, rowf_v, NPAD - 1)
    load_clamp(colS_hbm, colf_v, N - 1)

    def win(j, _):
        cb = _mo(j * GW)

        def cp(i2, _):
            sls = pl.ds(cb + _mo(i2 * LANES, LANES), LANES)
            sld = pl.ds(_mo(i2 * LANES, LANES), LANES)
            rowj_v[sld] = rowf_v[sls]
            colj_v[sld] = colf_v[sls]
            return 0
        lax.fori_loop(0, nv, cp, 0)

        pltpu.async_copy(ms_hbm.at[rowj_v], mswin, sem).wait()
        pltpu.sync_copy(mswin, msr_out.at[pl.ds(eb + cb, GW)])
        pltpu.async_copy(v_hbm.at[colj_v], vwin, sem).wait()
        pltpu.sync_copy(vwin, vc_out.at[pl.ds(eb + cb, GW)])
        return 0
    lax.fori_loop(0, nch, win, 0)


def _run_k3b(rowS, colS, ms_tab, v_tab):
    kern = pl.kernel(
        _k3b_gather,
        out_type=(_f32((EP2, D)), _f32((EP2, D))),
        mesh=_mesh,
        compiler_params=_SC_PARAMS,
        scratch_types=[
            pltpu.VMEM((EW,), jnp.int32),
            pltpu.VMEM((EW,), jnp.int32),
            pltpu.VMEM((GW,), jnp.int32),
            pltpu.VMEM((GW,), jnp.int32),
            pltpu.VMEM((GW, D), jnp.float32),
            pltpu.VMEM((GW, D), jnp.float32),
            pltpu.SemaphoreType.DMA,
        ],
        name="k3b_gather",
    )
    return kern(rowS, colS, ms_tab, v_tab)


def _k3c_unperm(inv_hbm, ue_hbm, out_hbm, invf_v, invj_v, uwin, sem):
    wid = _wid()
    eb = _mo(wid * EW)
    nch = EW // GW
    nv = GW // LANES

    pltpu.sync_copy(inv_hbm.at[pl.ds(eb, EW)], invf_v)

    def body(i, _):
        sl = pl.ds(_mo(i * LANES, LANES), LANES)
        invf_v[sl] = jnp.clip(invf_v[sl], 0, E - 1)
        return 0
    lax.fori_loop(0, EW // LANES, body, 0)

    def win(j, _):
        cb = _mo(j * GW)

        def cp(i2, _):
            sls = pl.ds(cb + _mo(i2 * LANES, LANES), LANES)
            sld = pl.ds(_mo(i2 * LANES, LANES), LANES)
            invj_v[sld] = invf_v[sls]
            return 0
        lax.fori_loop(0, nv, cp, 0)
        pltpu.async_copy(ue_hbm.at[invj_v], uwin, sem).wait()
        pltpu.sync_copy(uwin, out_hbm.at[pl.ds(eb + cb, GW)])
        return 0
    lax.fori_loop(0, nch, win, 0)


def _run_k3c(inv_p, ue_s):
    kern = pl.kernel(
        _k3c_unperm,
        out_type=_f32((EP2, D)),
        mesh=_mesh,
        compiler_params=_SC_PARAMS,
        scratch_types=[
            pltpu.VMEM((EW,), jnp.int32),
            pltpu.VMEM((GW,), jnp.int32),
            pltpu.VMEM((GW, D), jnp.float32),
            pltpu.SemaphoreType.DMA,
        ],
        name="k3c_unpermute",
    )
    return kern(inv_p, ue_s)


W6 = 80


def _k6_sums(ue_hbm, row_hbm, col_hbm, sum_out, sum_in, cnt_in,
             uew, uew2, riw, riw2, ones_w, zb, zc, tab_sh, cnt_sh,
             sem, sem2, sem3, sem4):
    c = lax.axis_index("c")
    s = lax.axis_index("s")
    eb = _mo(s * ECH)
    rows_per_tile = NPAD // NSUB

    nvz = D // LANES

    def zb_fill(i, _):
        zb[i // nvz, pl.ds(_mo((i % nvz) * LANES, LANES), LANES)] = (
            jnp.zeros((LANES,), jnp.float32))
        return 0
    lax.fori_loop(0, 64 * nvz, zb_fill, 0)

    def z(i, _):
        pltpu.sync_copy(
            zb, tab_sh.at[pl.ds(_mo(s * rows_per_tile + i * 64), 64)])
        return 0
    lax.fori_loop(0, rows_per_tile // 64, z, 0)

    def zc_fill(i, _):
        zc[pl.ds(_mo(i * LANES, LANES), LANES)] = jnp.zeros(
            (LANES,), jnp.float32)
        return 0
    lax.fori_loop(0, rows_per_tile // LANES, zc_fill, 0)
    pltpu.sync_copy(zc, cnt_sh.at[pl.ds(_mo(s * rows_per_tile),
                                        rows_per_tile)])

    def ones_fill(i, _):
        ones_w[pl.ds(_mo(i * LANES, LANES), LANES)] = jnp.ones(
            (LANES,), jnp.float32)
        return 0
    lax.fori_loop(0, W6 // LANES, ones_fill, 0)

    plsc.subcore_barrier()

    nwin6 = ECH // W6
    uews = [uew, uew2]
    riws = [riw, riw2]
    semsu = [sem, sem3]
    hs = [None, None]

    for w in range(nwin6):
        slot = w % 2
        base = _mo(eb + w * W6)
        hu = pltpu.async_copy(ue_hbm.at[pl.ds(base, W6)], uews[slot],
                              semsu[slot])

        @pl.when(c == 0)
        def _():
            pltpu.sync_copy(row_hbm.at[pl.ds(base, W6)], riws[slot])

        @pl.when(c == 1)
        def _():
            pltpu.sync_copy(col_hbm.at[pl.ds(base, W6)], riws[slot])

        hs[slot] = hu
        if w > 0:
            ps = (w - 1) % 2
            hs[ps].wait()
            pltpu.sync_copy(uews[ps], tab_sh.at[riws[ps]], add=True)
            pltpu.sync_copy(ones_w, cnt_sh.at[riws[ps]], add=True)
    ps = (nwin6 - 1) % 2
    hs[ps].wait()
    pltpu.sync_copy(uews[ps], tab_sh.at[riws[ps]], add=True)
    pltpu.sync_copy(ones_w, cnt_sh.at[riws[ps]], add=True)

    plsc.subcore_barrier()

    sl = pl.ds(_mo(s * rows_per_tile), rows_per_tile)
    slc = sl

    @pl.when(c == 0)
    def _():
        pltpu.sync_copy(tab_sh.at[sl], sum_out.at[sl])

    @pl.when(c == 1)
    def _():
        pltpu.sync_copy(tab_sh.at[sl], sum_in.at[sl])
        pltpu.sync_copy(cnt_sh.at[slc], cnt_in.at[slc])


def _run_k6(ue, row, col):
    kern = pl.kernel(
        _k6_sums,
        out_type=(_f32((NPAD, D)), _f32((NPAD, D)), _f32((NPAD,))),
        mesh=_mesh,
        compiler_params=_SC_PARAMS,
        scratch_types=[
            pltpu.VMEM((W6, D), jnp.float32),
            pltpu.VMEM((W6, D), jnp.float32),
            pltpu.VMEM((W6,), jnp.int32),
            pltpu.VMEM((W6,), jnp.int32),
            pltpu.VMEM((W6,), jnp.float32),
            pltpu.VMEM((64, D), jnp.float32),
            pltpu.VMEM((NPAD // NSUB,), jnp.float32),
            pltpu.VMEM_SHARED((NPAD, D), jnp.float32),
            pltpu.VMEM_SHARED((NPAD,), jnp.float32),
            pltpu.SemaphoreType.DMA,
            pltpu.SemaphoreType.DMA,
            pltpu.SemaphoreType.DMA,
            pltpu.SemaphoreType.DMA,
        ],
        name="k6_twin_sums",
    )
    return kern(ue, row, col)


def _k0a_pt(cn_ref, cr_ref, wt, bt, out):
    cnf = cn_ref[...]
    crf = cr_ref[...]
    a = jnp.repeat(cnf, 160, axis=0)
    b = jnp.tile(jnp.repeat(crf, 20, axis=0), (20, 1))
    cpart = jnp.tile(cnf, (160, 1))
    te = a + b + cpart
    nrm = jnp.sqrt(jnp.sum(te * te, axis=1, keepdims=True))
    te = te / (nrm + 1e-8)
    out[...] = te @ wt[...] + bt[...]


def _run_k0a(clip_node, clip_rel, wt, bt):
    return pl.pallas_call(
        _k0a_pt,
        out_shape=_f32((NCOMBO, D)),
    )(clip_node, clip_rel, wt, bt.reshape(1, D))


def _k0b_tabs(x_blk, pos_blk, wq, bq, wv, bv, row_tab, col_tab, v_tab):
    x = x_blk[...]
    p16 = pos_blk[...]
    q = x @ wq[...] + bq[...]
    v = x @ wv[...] + bv[...]
    zr = jnp.zeros((x.shape[0], TA - 2 * D - 16), jnp.float32)
    row_tab[...] = jnp.concatenate([q, x, p16, zr], axis=1)
    col_tab[...] = jnp.concatenate([x, v, p16, zr], axis=1)
    v_tab[...] = v


def _run_k0b(x, pos16, wq, bq, wv, bv):
    nb = N // 1000
    return pl.pallas_call(
        _k0b_tabs,
        grid=(nb,),
        in_specs=[
            pl.BlockSpec((1000, D), lambda i: (i, 0)),
            pl.BlockSpec((1000, 16), lambda i: (i, 0)),
            pl.BlockSpec((D, D), lambda i: (0, 0)),
            pl.BlockSpec((1, D), lambda i: (0, 0)),
            pl.BlockSpec((D, D), lambda i: (0, 0)),
            pl.BlockSpec((1, D), lambda i: (0, 0)),
        ],
        out_specs=[
            pl.BlockSpec((1000, TA), lambda i: (i, 0)),
            pl.BlockSpec((1000, TB), lambda i: (i, 0)),
            pl.BlockSpec((1000, D), lambda i: (i, 0)),
        ],
        out_shape=[_f32((N, TA)), _f32((N, TB)), _f32((N, D))],
    )(x, pos16, wq, bq.reshape(1, D), wv, bv.reshape(1, D))


BE = 640


def _k4_edge(a_ref, b_ref, t_ref, r_ref, mf_ref, ef_ref,
             wk, bk, wqk, wkk, wtk, b1k, w2s, b2s,
             dw1a, dw1b, db1, dw2, db2,
             w1a, w1b, w1c, w1d, eub1, euw2, eub2,
             lp_out, ue_out):
    a = a_ref[...]
    b = b_ref[...]
    q = a[:, 0:D]
    xr = a[:, D:2 * D]
    pr = a[:, 2 * D:2 * D + 16]
    xc = b[:, 0:D]
    pc = b[:, 2 * D:2 * D + 16]
    ef = ef_ref[...]
    t = t_ref[...]

    k = ef @ wk[...] + bk[...]
    h1 = jnp.maximum(
        q @ wqk[...] + k @ wkk[...] + t @ wtk[...] + b1k[...], 0.0)
    lg = h1 @ w2s[...] + b2s[...]

    diff = pr - pc
    dist = jnp.sqrt(jnp.sum(diff * diff, axis=1, keepdims=True) + 1e-12)
    hd = jnp.maximum(diff @ dw1a[...] + dist * dw1b[...] + db1[...], 0.0)
    dm = jax.nn.sigmoid(hd @ dw2[...] + db2[...])

    lp_out[...] = jnp.concatenate(
        [lg, dm, jnp.zeros((lg.shape[0], 15), jnp.float32)], axis=1)

    rev = r_ref[...] * mf_ref[...]
    hu = jnp.maximum(
        xr @ w1a[...] + xc @ w1b[...] + ef @ w1c[...] + rev @ w1d[...]
        + eub1[...], 0.0)
    ue_out[...] = hu @ euw2[...] + eub2[...]


def _run_k4(a, b, t, r, mf, efs, wk, bk, wqk, wkk, wtk, b1k, w2s, b2s,
            dw1a, dw1b, db1, dw2, db2, w1a, w1b, w1c, w1d, eub1, euw2, eub2):
    nb = E // BE
    full = lambda shape: pl.BlockSpec(shape, lambda i: (0, 0))
    return pl.pallas_call(
        _k4_edge,
        grid=(nb,),
        in_specs=[
            pl.BlockSpec((BE, TA), lambda i: (i, 0)),
            pl.BlockSpec((BE, TB), lambda i: (i, 0)),
            pl.BlockSpec((BE, D), lambda i: (i, 0)),
            pl.BlockSpec((BE, D), lambda i: (i, 0)),
            pl.BlockSpec((BE, 1), lambda i: (i, 0)),
            pl.BlockSpec((BE, D), lambda i: (i, 0)),
            full((D, D)), full((1, D)),
            full((D, 640)), full((D, 640)), full((D, 640)),
            full((1, 640)), full((640, 16)), full((1, 16)),
            full((16, 32)), full((1, 32)), full((1, 32)),
            full((32, 1)), full((1, 1)),
            full((D, 384)), full((D, 384)), full((D, 384)), full((D, 384)),
            full((1, 384)), full((384, D)), full((1, D)),
        ],
        out_specs=[
            pl.BlockSpec((BE, 32), lambda i: (i, 0)),
            pl.BlockSpec((BE, D), lambda i: (i, 0)),
        ],
        out_shape=[_f32((E, 32)), _f32((E, D))],
    )(a, b, t, r, mf, efs, wk, bk, wqk, wkk, wtk, b1k, w2s, b2s,
      dw1a, dw1b, db1, dw2, db2, w1a, w1b, w1c, w1d, eub1, euw2, eub2)


def _k4b_msg(lp_ref, msr_ref, vc_ref, selm, sels, prc, msg_out):
    lp = lp_ref[...]
    lg = lp[:, 0:16]
    dm = lp[:, 16:17]
    msr = msr_ref[...]
    m = msr @ selm[...]
    s = msr @ sels[...]
    p = jnp.exp(lg - m) / (s + 1e-9)
    alpha = (p @ prc[...]) * dm
    msg_out[...] = vc_ref[...] * alpha


def _run_k4b(lp, msr, vc, selm, sels, prc):
    nb = E // BE
    full = lambda shape: pl.BlockSpec(shape, lambda i: (0, 0))
    return pl.pallas_call(
        _k4b_msg,
        grid=(nb,),
        in_specs=[
            pl.BlockSpec((BE, 32), lambda i: (i, 0)),
            pl.BlockSpec((BE, D), lambda i: (i, 0)),
            pl.BlockSpec((BE, D), lambda i: (i, 0)),
            full((D, 16)), full((D, 16)), full((16, D)),
        ],
        out_specs=pl.BlockSpec((BE, D), lambda i: (i, 0)),
        out_shape=_f32((E, D)),
    )(lp, msr, vc, selm, sels, prc)


def _k7_node(x_ref, agg_ref, so_ref, si_ref, co_ref, ci_ref,
             nw1a, nw1b, nb1, nw2, nb2, eawa, eawb, eab, out):
    x = x_ref[...]
    agg = agg_ref[...]
    h = jnp.maximum(x @ nw1a[...] + agg @ nw1b[...] + nb1[...], 0.0)
    un = h @ nw2[...] + nb2[...]
    co = jnp.maximum(co_ref[...], 1.0)
    ci = jnp.maximum(ci_ref[...], 1.0)
    om = so_ref[...] / co
    im = si_ref[...] / ci
    gate = jax.nn.sigmoid(om @ eawa[...] + im @ eawb[...] + eab[...])
    out[...] = un * gate


def _run_k7(x, agg, so, si, co, ci, nw1a, nw1b, nb1, nw2, nb2,
            eawa, eawb, eab):
    nb = N // 1000
    full = lambda shape: pl.BlockSpec(shape, lambda i: (0, 0))
    return pl.pallas_call(
        _k7_node,
        grid=(nb,),
        in_specs=[
            pl.BlockSpec((1000, D), lambda i: (i, 0)),
            pl.BlockSpec((1000, D), lambda i: (i, 0)),
            pl.BlockSpec((1000, D), lambda i: (i, 0)),
            pl.BlockSpec((1000, D), lambda i: (i, 0)),
            pl.BlockSpec((1000, 1), lambda i: (i, 0)),
            pl.BlockSpec((1000, 1), lambda i: (i, 0)),
            full((D, 256)), full((D, 256)), full((1, 256)),
            full((256, D)), full((1, D)),
            full((D, D)), full((D, D)), full((1, D)),
        ],
        out_specs=pl.BlockSpec((1000, D), lambda i: (i, 0)),
        out_shape=_f32((N, D)),
    )(x, agg, so, si, co, ci, nw1a, nw1b, nb1, nw2, nb2, eawa, eawb, eab)


def kernel(x, edge_feature, node_positions, params, edge_index,
           gt_rel_label, gt_obj_label):
    p = params
    row = edge_index[0].astype(jnp.int32)
    col = edge_index[1].astype(jnp.int32)
    obj = gt_obj_label.astype(jnp.int32)
    rel = gt_rel_label.astype(jnp.int32)
    pos16 = jnp.pad(node_positions, ((0, 0), (0, 13)))

    eye = jnp.eye(H, dtype=jnp.float32)
    wqk = jnp.concatenate([jnp.kron(eye, p['a3W1'][:DH]),
                           jnp.kron(eye, p['atW1'][:DH])], axis=1)
    wkk = jnp.concatenate([jnp.kron(eye, p['a3W1'][DH:2 * DH]),
                           jnp.kron(eye, p['atW1'][DH:2 * DH])], axis=1)
    wtk = jnp.concatenate([jnp.zeros((D, 256), jnp.float32),
                           jnp.kron(eye, p['atW1'][2 * DH:3 * DH])], axis=1)
    b1k = jnp.concatenate([jnp.tile(p['a3b1'], H),
                           jnp.tile(p['atb1'], H)]).reshape(1, 640)
    w2s_a = jnp.kron(eye, p['a3W2'].sum(axis=1)[:, None])
    w2s_t = jnp.kron(eye, p['atW2'].sum(axis=1)[:, None])
    w2s = jnp.concatenate([
        jnp.concatenate([w2s_a, jnp.zeros((256, 8), jnp.float32)], axis=1),
        jnp.concatenate([jnp.zeros((384, 8), jnp.float32), w2s_t], axis=1),
    ], axis=0) / TEMP
    b2s = jnp.concatenate([
        jnp.full((8,), p['a3b2'].sum(), jnp.float32),
        jnp.full((8,), p['atb2'].sum(), jnp.float32)]).reshape(1, 16) / TEMP

    dw1a = jnp.pad(p['dW1'][:3], ((0, 13), (0, 0)))
    dw1b = p['dW1'][3].reshape(1, 32)
    db1 = p['db1'].reshape(1, 32)
    dw2 = p['dW2']
    db2 = p['db2'].reshape(1, 1)

    w1a = p['euW1'][0:D]
    w1b = p['euW1'][D:2 * D]
    w1c = p['euW1'][2 * D:3 * D]
    w1d = p['euW1'][3 * D:4 * D]
    eub1 = p['eub1'].reshape(1, 384)
    eub2 = p['eub2'].reshape(1, D)

    nw1a = p['nuW1'][0:D]
    nw1b = p['nuW1'][D:2 * D]
    nb1 = p['nub1'].reshape(1, 256)
    nb2 = p['nub2'].reshape(1, D)
    eawa = p['eaW'][0:D]
    eawb = p['eaW'][D:2 * D]
    eab = p['eab'].reshape(1, D)

    selm = jnp.zeros((D, 16), jnp.float32).at[:16, :].set(jnp.eye(16))
    sels = jnp.zeros((D, 16), jnp.float32).at[16:32, :].set(jnp.eye(16))
    prc = jnp.concatenate([jnp.kron(eye, jnp.ones((1, DH), jnp.float32)),
                           jnp.kron(eye, jnp.ones((1, DH), jnp.float32))],
                          axis=0) * 0.5

    pt_tab = _run_k0a(p['clip_node'], p['clip_rel'], p['Wt'], p['bt'])
    row_tab, col_tab, v_tab = _run_k0b(x, pos16, p['Wq'], p['bq'],
                                       p['Wv'], p['bv'])
    rowS, colS, idxS, cnt, off, inv2d = _run_k1(row, col)
    ridxS, matchS = _run_k2(rowS, colS, off, cnt, idxS)
    a_g, b_g, t_g, r_g, efs = _run_k3(rowS, colS, idxS, obj, rel, ridxS,
                                      row_tab, col_tab, pt_tab, edge_feature)

    mf = matchS.astype(jnp.float32).reshape(EP2, 1)
    lp, ue_s = _run_k4(a_g, b_g, t_g, r_g, mf, efs,
                       p['Wk'], p['bk'].reshape(1, D), wqk, wkk, wtk, b1k,
                       w2s, b2s, dw1a, dw1b, db1, dw2, db2,
                       w1a, w1b, w1c, w1d, eub1, p['euW2'], eub2)

    ms_tab = _run_k5a(lp.reshape(E * 32), rowS, off).reshape(NPAD, D)
    msr, vc = _run_k3b(rowS, colS, ms_tab, v_tab)
    msg = _run_k4b(lp, msr, vc, selm, sels, prc)
    agg = _run_k5b(msg.reshape(E * D), rowS, off).reshape(NPAD, D)

    inv_p = jnp.pad(inv2d[:E], (0, EP2 - E))
    ue = _run_k3c(inv_p, ue_s)[:E]

    sum_out, sum_in, cnt_in = _run_k6(ue, row, col)

    updated_node = _run_k7(
        x, agg[:N], sum_out[:N], sum_in[:N],
        cnt[:N].astype(jnp.float32).reshape(N, 1), cnt_in[:N].reshape(N, 1),
        nw1a, nw1b, nb1, p['nuW2'], nb2, eawa, eawb, eab)

    return updated_node, ue


# speedup vs baseline: 2.6627x; 2.6415x over previous
import jax
import jax.numpy as jnp
from jax import lax
from jax.experimental import pallas as pl
from jax.experimental.pallas import tpu as pltpu
from jax.experimental.pallas import tpu_sc as plsc

N = 10000
E = 160000
D = 128
H = 8
DH = 16
TEMP = 4.0

NCORE = 2
NSUB = 16
NW = NCORE * NSUB
LANES = 16

RPT = 320
NPAD = NW * RPT
RPC = NPAD // 2
ECH = E // NSUB
EW = 5120
EP2 = NW * EW
IDXW = 80
GW = 128
GW3 = 64
TA = 384
TB = 384
NCOMBO = 3200
W5 = 512

_mesh = plsc.VectorSubcoreMesh(core_axis_name="c", subcore_axis_name="s")
_SC_PARAMS = pltpu.CompilerParams(needs_layout_passes=False)


def _f32(shape):
    return jax.ShapeDtypeStruct(shape, jnp.float32)


def _i32(shape):
    return jax.ShapeDtypeStruct(shape, jnp.int32)


def _wid():
    return lax.axis_index("s") * NCORE + lax.axis_index("c")


def _mo(x, n=8):
    return pl.multiple_of(x, n)


def _zero_i32(ref, n):
    def body(i, _):
        ref[pl.ds(i * LANES, LANES)] = jnp.zeros((LANES,), jnp.int32)
        return 0
    lax.fori_loop(0, n // LANES, body, 0)


def _fill2d_f32(ref, rows, width, val):
    nv = width // LANES

    def body(i, _):
        ref[i // nv, pl.ds((i % nv) * LANES, LANES)] = jnp.full(
            (LANES,), val, jnp.float32)
        return 0
    lax.fori_loop(0, rows * nv, body, 0)


def _clamp_to_2d(src_flat, dst2d, total, width, lo, hi):
    nv = width // LANES

    def body(i, _):
        v = src_flat[pl.ds(i * LANES, LANES)]
        dst2d[i // nv, pl.ds((i % nv) * LANES, LANES)] = jnp.clip(v, lo, hi)
        return 0
    lax.fori_loop(0, total // LANES, body, 0)


def _k1_sort(row_hbm, col_hbm, rowS, colS, idxS, cnt_hbm, off_hbm, inv_hbm,
             rows_v, payl_v, rank_v, hist_v, tmp_v, base_v, tot_v,
             off_v, posj_v, rowj_v, colj_v, idxj_v, grid_sh):
    c = lax.axis_index("c")
    s = lax.axis_index("s")
    rlo = c * RPC
    eb = _mo(s * ECH)

    pltpu.sync_copy(row_hbm.at[pl.ds(eb, ECH)], rows_v)
    _zero_i32(hist_v, RPC)

    def hist_body(i, _):
        sl = pl.ds(_mo(i * LANES, LANES), LANES)
        r = rows_v[sl]
        m = (r >= rlo) & (r < rlo + RPC)
        rl = jnp.where(m, r - rlo, 0)
        occ, last = plsc.scan_count(rl, mask=m)
        base = plsc.load_gather(hist_v, [rl])
        rank_v[sl] = base + occ - 1
        plsc.store_scatter(hist_v, [rl], base + occ, mask=last & m)
        return 0
    lax.fori_loop(0, ECH // LANES, hist_body, 0)

    pltpu.sync_copy(hist_v, grid_sh.at[pl.ds(_mo(s * RPC), RPC)])
    plsc.subcore_barrier()

    _zero_i32(base_v, RPC)
    _zero_i32(tot_v, RPC)
    for sp in range(NSUB):
        pltpu.sync_copy(grid_sh.at[pl.ds(sp * RPC, RPC)], tmp_v)
        use = jnp.int32(sp) < s

        def acc_body(i, _):
            sl = pl.ds(_mo(i * LANES, LANES), LANES)
            tv = tmp_v[sl]
            base_v[sl] = base_v[sl] + jnp.where(use, tv, 0)
            tot_v[sl] = tot_v[sl] + tv
            return 0
        lax.fori_loop(0, RPC // LANES, acc_body, 0)

    def sum_body(i, acc):
        return acc + jnp.sum(tot_v[pl.ds(_mo(i * LANES, LANES), LANES)])
    core_total = lax.fori_loop(0, RPC // LANES, sum_body, jnp.int32(0))
    core_base = jnp.where(c == 0, jnp.int32(0), jnp.int32(E) - core_total)

    def scan_body(i, carry):
        sl = pl.ds(_mo(i * LANES, LANES), LANES)
        v = tot_v[sl]
        cs = plsc.cumsum(v)
        off_v[sl] = carry + core_base + cs - v
        return carry + jnp.sum(v)
    lax.fori_loop(0, RPC // LANES, scan_body, jnp.int32(0))

    sl = pl.ds(_mo(s * RPT), RPT)
    pltpu.sync_copy(tot_v.at[sl], cnt_hbm.at[pl.ds(_mo(c * RPC + s * RPT),
                                                   RPT)])
    pltpu.sync_copy(off_v.at[sl], off_hbm.at[pl.ds(_mo(c * RPC + s * RPT),
                                                   RPT)])

    pltpu.sync_copy(col_hbm.at[pl.ds(eb, ECH)], payl_v)

    wid = _wid()

    def chunk(j, _):
        cb = _mo(j * IDXW)
        dump = (jnp.int32(E)
                + lax.rem(wid * 37 + j, jnp.int32((EP2 - E) // IDXW))
                * IDXW)

        def vec(k2, _):
            sl_src = pl.ds(cb + _mo(k2 * LANES, LANES), LANES)
            sl_dst = pl.ds(_mo(k2 * LANES, LANES), LANES)
            r = rows_v[sl_src]
            m = (r >= rlo) & (r < rlo + RPC)
            rl = jnp.where(m, r - rlo, 0)
            pos = (plsc.load_gather(off_v, [rl])
                   + plsc.load_gather(base_v, [rl])
                   + rank_v[sl_src])
            dmp = dump + k2 * LANES + lax.iota(jnp.int32, LANES)
            pos = jnp.where(m, pos, dmp)
            posj_v[sl_dst] = pos
            rowj_v[sl_dst] = r
            colj_v[sl_dst] = payl_v[sl_src]
            idxj_v[sl_dst] = jnp.where(
                m, eb + cb + k2 * LANES + lax.iota(jnp.int32, LANES), dmp)
            return 0
        lax.fori_loop(0, IDXW // LANES, vec, 0)
        pltpu.sync_copy(rowj_v, rowS.at[posj_v])
        pltpu.sync_copy(colj_v, colS.at[posj_v])
        pltpu.sync_copy(idxj_v, idxS.at[posj_v])
        pltpu.sync_copy(posj_v, inv_hbm.at[idxj_v])
        return 0
    lax.fori_loop(0, ECH // IDXW, chunk, 0)


def _run_k1(row, col):
    kern = pl.kernel(
        _k1_sort,
        out_type=(_i32((EP2,)), _i32((EP2,)), _i32((EP2,)),
                  _i32((NPAD,)), _i32((NPAD,)), _i32((EP2,))),
        mesh=_mesh,
        compiler_params=_SC_PARAMS,
        scratch_types=[
            pltpu.VMEM((ECH,), jnp.int32),
            pltpu.VMEM((ECH,), jnp.int32),
            pltpu.VMEM((ECH,), jnp.int32),
            pltpu.VMEM((RPC,), jnp.int32),
            pltpu.VMEM((RPC,), jnp.int32),
            pltpu.VMEM((RPC,), jnp.int32),
            pltpu.VMEM((RPC,), jnp.int32),
            pltpu.VMEM((RPC,), jnp.int32),
            pltpu.VMEM((IDXW,), jnp.int32),
            pltpu.VMEM((IDXW,), jnp.int32),
            pltpu.VMEM((IDXW,), jnp.int32),
            pltpu.VMEM((IDXW,), jnp.int32),
            pltpu.VMEM_SHARED((NSUB * RPC,), jnp.int32),
        ],
        name="k1_sort",
    )
    return kern(row, col)


def _k2_walk(rowS_hbm, colS_hbm, off_hbm, cnt_hbm, idxS,
             ridx_hbm, match_hbm,
             tgt_v, ptr_v, end_v, hitp_v, vals_v, res_v, flj_v, flj2_v,
             sem, sem2):
    wid = _wid()
    eb = _mo(wid * EW)
    nch = EW // GW

    pltpu.sync_copy(rowS_hbm.at[pl.ds(eb, EW)], tgt_v)
    pltpu.sync_copy(colS_hbm.at[pl.ds(eb, EW)], vals_v)

    def g_off(j, _):
        cb = _mo(j * GW)

        def cl(i2, _):
            v = vals_v[pl.ds(cb + _mo(i2 * LANES, LANES), LANES)]
            flj_v[pl.ds(_mo(i2 * LANES, LANES), LANES)] = jnp.clip(
                v, 0, NPAD - 1)
            return 0
        lax.fori_loop(0, GW // LANES, cl, 0)
        pltpu.async_copy(off_hbm.at[flj_v], ptr_v.at[pl.ds(cb, GW)],
                         sem).wait()
        pltpu.async_copy(cnt_hbm.at[flj_v], end_v.at[pl.ds(cb, GW)],
                         sem).wait()
        return 0
    lax.fori_loop(0, nch, g_off, 0)

    def init_body(i, _):
        sl = pl.ds(_mo(i * LANES, LANES), LANES)
        end_v[sl] = ptr_v[sl] + end_v[sl]
        hitp_v[sl] = jnp.full((LANES,), -1, jnp.int32)
        return 0
    lax.fori_loop(0, EW // LANES, init_body, 0)

    def walk_cond(carry):
        return carry > 0

    def walk_body(carry):
        def fill(j, dst):
            for i2 in range(GW // LANES):
                p = ptr_v[pl.ds(j * GW + i2 * LANES, LANES)]
                ok = (p >= 0) & (p < E)
                dst[pl.ds(i2 * LANES, LANES)] = jnp.where(ok, p, E)

        def update(j, n_act):
            for i2 in range(GW // LANES):
                sl = pl.ds(j * GW + i2 * LANES, LANES)
                p = ptr_v[sl]
                active = (p < end_v[sl]) & (hitp_v[sl] < 0)
                hit = active & (vals_v[sl] == tgt_v[sl])
                hitp_v[sl] = jnp.where(hit, p, hitp_v[sl])
                still = active & ~hit
                ptr_v[sl] = jnp.where(still, p + 1, p)
                n_act = n_act + jnp.sum(jnp.where(still, 1, 0))
            return n_act

        fls = [flj_v, flj2_v]
        sems = [sem, sem2]
        handles = [None, None]
        n_act = jnp.int32(0)
        for j in range(nch):
            fill(j, fls[j % 2])
            handles[j % 2] = pltpu.async_copy(
                colS_hbm.at[fls[j % 2]],
                vals_v.at[pl.ds(j * GW, GW)], sems[j % 2])
            if j > 0:
                handles[(j - 1) % 2].wait()
                n_act = update(j - 1, n_act)
        handles[(nch - 1) % 2].wait()
        n_act = update(nch - 1, n_act)
        return n_act

    lax.while_loop(walk_cond, walk_body, jnp.int32(1))

    def g_res(j, _):
        cb = _mo(j * GW)

        def cl(i2, _):
            p = hitp_v[pl.ds(cb + _mo(i2 * LANES, LANES), LANES)]
            flj_v[pl.ds(_mo(i2 * LANES, LANES), LANES)] = jnp.where(
                p >= 0, p, E)
            return 0
        lax.fori_loop(0, GW // LANES, cl, 0)
        pltpu.async_copy(idxS.at[flj_v], vals_v.at[pl.ds(cb, GW)],
                         sem).wait()
        return 0
    lax.fori_loop(0, nch, g_res, 0)

    def fin(i, _):
        sl = pl.ds(_mo(i * LANES, LANES), LANES)
        ok = hitp_v[sl] >= 0
        res_v[sl] = jnp.clip(jnp.where(ok, vals_v[sl], 0), 0, E - 1)
        ptr_v[sl] = jnp.where(ok, 1, 0)
        return 0
    lax.fori_loop(0, EW // LANES, fin, 0)

    pltpu.sync_copy(res_v, ridx_hbm.at[pl.ds(eb, EW)])
    pltpu.sync_copy(ptr_v, match_hbm.at[pl.ds(eb, EW)])


def _run_k2(rowS, colS, off, cnt, idxS):
    kern = pl.kernel(
        _k2_walk,
        out_type=(_i32((EP2,)), _i32((EP2,))),
        mesh=_mesh,
        compiler_params=_SC_PARAMS,
        scratch_types=[
            pltpu.VMEM((EW,), jnp.int32),
            pltpu.VMEM((EW,), jnp.int32),
            pltpu.VMEM((EW,), jnp.int32),
            pltpu.VMEM((EW,), jnp.int32),
            pltpu.VMEM((EW,), jnp.int32),
            pltpu.VMEM((EW,), jnp.int32),
            pltpu.VMEM((GW,), jnp.int32),
            pltpu.VMEM((GW,), jnp.int32),
            pltpu.SemaphoreType.DMA,
            pltpu.SemaphoreType.DMA,
        ],
        name="k2_revlookup",
    )
    return kern(rowS, colS, off, cnt, idxS)


def _k3_gather(rowS_hbm, colS_hbm, idxS_hbm, obj_hbm, rel_hbm, ridx_hbm,
               row_tab, col_tab, pt_tab, ef_hbm,
               a_out, b_out, t_out, r_out, efs_out,
               rowf_v, colf_v, idxf_v, ridxf_v,
               rowj_v, colj_v, idxj_v, ridxj_v, comboj_v, objcj_v, relj_v,
               awin, bwin, twin, rwin, ewin, sem, sem2, sem3, sem4, sem5):
    wid = _wid()
    eb = _mo(wid * EW)
    nch = EW // GW3
    nv = GW3 // LANES

    def load_clamp(src_hbm, dst, hi):
        pltpu.sync_copy(src_hbm.at[pl.ds(eb, EW)], dst)

        def body(i, _):
            sl = pl.ds(_mo(i * LANES, LANES), LANES)
            dst[sl] = jnp.clip(dst[sl], 0, hi)
            return 0
        lax.fori_loop(0, EW // LANES, body, 0)

    load_clamp(rowS_hbm, rowf_v, N - 1)
    load_clamp(colS_hbm, colf_v, N - 1)
    load_clamp(idxS_hbm, idxf_v, E - 1)
    load_clamp(ridx_hbm, ridxf_v, E - 1)

    def chunk(j, _):
        cb = _mo(j * GW3)

        def cp(i2, _):
            sls = pl.ds(cb + _mo(i2 * LANES, LANES), LANES)
            sld = pl.ds(_mo(i2 * LANES, LANES), LANES)
            rowj_v[sld] = rowf_v[sls]
            colj_v[sld] = colf_v[sls]
            idxj_v[sld] = idxf_v[sls]
            ridxj_v[sld] = ridxf_v[sls]
            return 0
        lax.fori_loop(0, nv, cp, 0)

        pltpu.async_copy(obj_hbm.at[rowj_v], comboj_v, sem).wait()
        pltpu.async_copy(obj_hbm.at[colj_v], objcj_v, sem).wait()
        pltpu.async_copy(rel_hbm.at[idxj_v], relj_v, sem).wait()

        def mix(i2, _):
            sl = pl.ds(_mo(i2 * LANES, LANES), LANES)
            comboj_v[sl] = (comboj_v[sl] * 160 + relj_v[sl] * 20
                            + objcj_v[sl])
            return 0
        lax.fori_loop(0, nv, mix, 0)

        ob = pl.ds(eb + cb, GW3)
        ha = pltpu.async_copy(row_tab.at[rowj_v], awin, sem)
        hb = pltpu.async_copy(col_tab.at[colj_v], bwin, sem2)
        ht = pltpu.async_copy(pt_tab.at[comboj_v], twin, sem3)
        he = pltpu.async_copy(ef_hbm.at[idxj_v], ewin, sem4)
        hr = pltpu.async_copy(ef_hbm.at[ridxj_v], rwin, sem5)
        ha.wait()
        pltpu.sync_copy(awin, a_out.at[ob])
        hb.wait()
        pltpu.sync_copy(bwin, b_out.at[ob])
        ht.wait()
        pltpu.sync_copy(twin, t_out.at[ob])
        he.wait()
        pltpu.sync_copy(ewin, efs_out.at[ob])
        hr.wait()
        pltpu.sync_copy(rwin, r_out.at[ob])
        return 0
    lax.fori_loop(0, nch, chunk, 0)


def _run_k3(rowS, colS, idxS, obj, rel, ridx, row_tab, col_tab, pt_tab,
            ef):
    kern = pl.kernel(
        _k3_gather,
        out_type=(_f32((EP2, TA)), _f32((EP2, TB)), _f32((EP2, D)),
                  _f32((EP2, D)), _f32((EP2, D))),
        mesh=_mesh,
        compiler_params=_SC_PARAMS,
        scratch_types=[
            pltpu.VMEM((EW,), jnp.int32),
            pltpu.VMEM((EW,), jnp.int32),
            pltpu.VMEM((EW,), jnp.int32),
            pltpu.VMEM((EW,), jnp.int32),
            pltpu.VMEM((GW3,), jnp.int32),
            pltpu.VMEM((GW3,), jnp.int32),
            pltpu.VMEM((GW3,), jnp.int32),
            pltpu.VMEM((GW3,), jnp.int32),
            pltpu.VMEM((GW3,), jnp.int32),
            pltpu.VMEM((GW3,), jnp.int32),
            pltpu.VMEM((GW3,), jnp.int32),
            pltpu.VMEM((GW3, TA), jnp.float32),
            pltpu.VMEM((GW3, TB), jnp.float32),
            pltpu.VMEM((GW3, D), jnp.float32),
            pltpu.VMEM((GW3, D), jnp.float32),
            pltpu.VMEM((GW3, D), jnp.float32),
            pltpu.SemaphoreType.DMA,
            pltpu.SemaphoreType.DMA,
            pltpu.SemaphoreType.DMA,
            pltpu.SemaphoreType.DMA,
            pltpu.SemaphoreType.DMA,
        ],
        name="k3_gather",
    )
    return kern(rowS, colS, idxS, obj, rel, ridx, row_tab, col_tab,
                pt_tab, ef)


def _k5a_ms(lpf_hbm, rowS_hbm, off_hbm, ms_out,
            m_tab, s_tab, ms_buf, rows_w, lpw, ob):
    wid = _wid()
    rlo = wid * RPT

    pltpu.sync_copy(off_hbm.at[pl.ds(_mo(rlo), LANES)], ob)
    start = ob[pl.ds(0, LANES)][0]
    is_last = wid == NW - 1
    nxt = _mo(jnp.where(is_last, NPAD - LANES, rlo + RPT))
    pltpu.sync_copy(off_hbm.at[pl.ds(nxt, LANES)], ob)
    end = jnp.where(is_last, jnp.int32(E), ob[pl.ds(0, LANES)][0])

    abase = start - lax.rem(start, jnp.int32(8))
    nwin = (end - abase + (W5 - 1)) // W5

    def fill(ref, n, val):
        def body(i, _):
            ref[pl.ds(_mo(i * LANES, LANES), LANES)] = jnp.full(
                (LANES,), val, jnp.float32)
            return 0
        lax.fori_loop(0, n // LANES, body, 0)

    fill(m_tab, RPT * LANES, -jnp.inf)
    fill(s_tab, RPT * LANES, 0.0)

    def win_common(w):
        base_u = abase + w * W5
        base = _mo(jnp.minimum(base_u, jnp.int32(E - W5)))
        pltpu.sync_copy(lpf_hbm.at[pl.ds(_mo(base * 32), W5 * 32)], lpw)
        pltpu.sync_copy(rowS_hbm.at[pl.ds(base, W5)], rows_w)
        return base_u, base

    def win_a(w, _):
        base_u, base = win_common(w)

        def edge(i16, _):
            rvec = rows_w[pl.ds(_mo(i16 * LANES, LANES), LANES)]
            for k in range(LANES):
                i = i16 * LANES + k
                rloc = rvec[k] - rlo
                pp = base + i
                ok = ((rloc >= 0) & (rloc < RPT) & (pp >= start)
                      & (pp < end) & (pp >= base_u))
                rc = jnp.clip(rloc, 0, RPT - 1)
                lv = lpw[pl.ds(_mo(i * 32, LANES), LANES)]
                msl = pl.ds(_mo(rc * LANES, LANES), LANES)
                cur = m_tab[msl]
                m_tab[msl] = jnp.where(ok, jnp.maximum(cur, lv), cur)
            return 0
        lax.fori_loop(0, W5 // LANES, edge, 0)
        return 0
    lax.fori_loop(0, nwin, win_a, 0)

    def win_b(w, _):
        base_u, base = win_common(w)

        def edge(i16, _):
            rvec = rows_w[pl.ds(_mo(i16 * LANES, LANES), LANES)]
            for k in range(LANES):
                i = i16 * LANES + k
                rloc = rvec[k] - rlo
                pp = base + i
                ok = ((rloc >= 0) & (rloc < RPT) & (pp >= start)
                      & (pp < end) & (pp >= base_u))
                rc = jnp.clip(rloc, 0, RPT - 1)
                lv = lpw[pl.ds(_mo(i * 32, LANES), LANES)]
                msl = pl.ds(_mo(rc * LANES, LANES), LANES)
                ev = jnp.exp(lv - m_tab[msl])
                s_tab[msl] = s_tab[msl] + jnp.where(ok, ev, 0.0)
            return 0
        lax.fori_loop(0, W5 // LANES, edge, 0)
        return 0
    lax.fori_loop(0, nwin, win_b, 0)

    fill(ms_buf, RPT * D, 0.0)

    def pack(r, _):
        sl = pl.ds(_mo(r * LANES, LANES), LANES)
        ms_buf[pl.ds(_mo(r * D), LANES)] = m_tab[sl]
        ms_buf[pl.ds(_mo(r * D + LANES), LANES)] = s_tab[sl]
        return 0
    lax.fori_loop(0, RPT, pack, 0)

    pltpu.sync_copy(ms_buf, ms_out.at[pl.ds(_mo(rlo * D), RPT * D)])


def _run_k5a(lpf, rowS, off):
    kern = pl.kernel(
        _k5a_ms,
        out_type=_f32((NPAD * D,)),
        mesh=_mesh,
        compiler_params=_SC_PARAMS,
        scratch_types=[
            pltpu.VMEM((RPT * LANES,), jnp.float32),
            pltpu.VMEM((RPT * LANES,), jnp.float32),
            pltpu.VMEM((RPT * D,), jnp.float32),
            pltpu.VMEM((W5,), jnp.int32),
            pltpu.VMEM((W5 * 32,), jnp.float32),
            pltpu.VMEM((LANES,), jnp.int32),
        ],
        name="k5a_softmax_stats",
    )
    return kern(lpf, rowS, off)


NEGBIG = -3.4e38


def _k5b_agg(msgf_hbm, rowS_hbm, off_hbm, agg_out,
             agg_tab, rows_w, mw, ob):
    wid = _wid()
    rlo = wid * RPT

    pltpu.sync_copy(off_hbm.at[pl.ds(_mo(rlo), LANES)], ob)
    start = ob[pl.ds(0, LANES)][0]
    is_last = wid == NW - 1
    nxt = _mo(jnp.where(is_last, NPAD - LANES, rlo + RPT))
    pltpu.sync_copy(off_hbm.at[pl.ds(nxt, LANES)], ob)
    end = jnp.where(is_last, jnp.int32(E), ob[pl.ds(0, LANES)][0])

    abase = start - lax.rem(start, jnp.int32(8))
    nwin = (end - abase + (W5 - 1)) // W5

    def fill(ref, n, val):
        def body(i, _):
            ref[pl.ds(_mo(i * LANES, LANES), LANES)] = jnp.full(
                (LANES,), val, jnp.float32)
            return 0
        lax.fori_loop(0, n // LANES, body, 0)

    fill(agg_tab, RPT * D, NEGBIG)

    def win(w, _):
        base_u = abase + w * W5
        base = _mo(jnp.minimum(base_u, jnp.int32(E - W5)))
        pltpu.sync_copy(msgf_hbm.at[pl.ds(_mo(base * D), W5 * D)], mw)
        pltpu.sync_copy(rowS_hbm.at[pl.ds(base, W5)], rows_w)

        def edge(i16, _):
            rvec = rows_w[pl.ds(_mo(i16 * LANES, LANES), LANES)]
            for k in range(LANES):
                i = i16 * LANES + k
                rloc = rvec[k] - rlo
                pp = base + i
                ok = ((rloc >= 0) & (rloc < RPT) & (pp >= start)
                      & (pp < end) & (pp >= base_u))
                rc = jnp.clip(rloc, 0, RPT - 1)
                for h in range(D // LANES):
                    asl = pl.ds(_mo(rc * D + h * LANES, LANES), LANES)
                    cur = agg_tab[asl]
                    v = mw[pl.ds(_mo(i * D + h * LANES, LANES), LANES)]
                    agg_tab[asl] = jnp.where(ok, jnp.maximum(cur, v), cur)
            return 0
        lax.fori_loop(0, W5 // LANES, edge, 0)
        return 0
    lax.fori_loop(0, nwin, win, 0)

    def fix(i, _):
        sl = pl.ds(_mo(i * LANES, LANES), LANES)
        v = agg_tab[sl]
        agg_tab[sl] = jnp.where(v <= jnp.float32(-3.0e38), 0.0, v)
        return 0
    lax.fori_loop(0, RPT * (D // LANES), fix, 0)

    pltpu.sync_copy(agg_tab, agg_out.at[pl.ds(_mo(rlo * D), RPT * D)])


def _run_k5b(msgf, rowS, off):
    kern = pl.kernel(
        _k5b_agg,
        out_type=_f32((NPAD * D,)),
        mesh=_mesh,
        compiler_params=_SC_PARAMS,
        scratch_types=[
            pltpu.VMEM((RPT * D,), jnp.float32),
            pltpu.VMEM((W5,), jnp.int32),
            pltpu.VMEM((W5 * D,), jnp.float32),
            pltpu.VMEM((LANES,), jnp.int32),
        ],
        name="k5b_aggmax",
    )
    return kern(msgf, rowS, off)


def _k3b_gather(rowS_hbm, colS_hbm, ms_hbm, v_hbm,
                msr_out, vc_out,
                rowf_v, colf_v, rowj_v, colj_v, mswin, vwin, sem):
    wid = _wid()
    eb = _mo(wid * EW)
    nch = EW // GW
    nv = GW // LANES

    def load_clamp(src_hbm, dst, hi):
        pltpu.sync_copy(src_hbm.at[pl.ds(eb, EW)], dst)

        def body(i, _):
            sl = pl.ds(_mo(i * LANES, LANES), LANES)
            dst[sl] = jnp.clip(dst[sl], 0, hi)
            return 0
        lax.fori_loop(0, EW // LANES, body, 0)

    load_clamp(rowS_hbm, rowf_v, NPAD - 1)
    load_clamp(colS_hbm, colf_v, N - 1)

    def win(j, _):
        cb = _mo(j * GW)

        def cp(i2, _):
            sls = pl.ds(cb + _mo(i2 * LANES, LANES), LANES)
            sld = pl.ds(_mo(i2 * LANES, LANES), LANES)
            rowj_v[sld] = rowf_v[sls]
            colj_v[sld] = colf_v[sls]
            return 0
        lax.fori_loop(0, nv, cp, 0)

        pltpu.async_copy(ms_hbm.at[rowj_v], mswin, sem).wait()
        pltpu.sync_copy(mswin, msr_out.at[pl.ds(eb + cb, GW)])
        pltpu.async_copy(v_hbm.at[colj_v], vwin, sem).wait()
        pltpu.sync_copy(vwin, vc_out.at[pl.ds(eb + cb, GW)])
        return 0
    lax.fori_loop(0, nch, win, 0)


def _run_k3b(rowS, colS, ms_tab, v_tab):
    kern = pl.kernel(
        _k3b_gather,
        out_type=(_f32((EP2, D)), _f32((EP2, D))),
        mesh=_mesh,
        compiler_params=_SC_PARAMS,
        scratch_types=[
            pltpu.VMEM((EW,), jnp.int32),
            pltpu.VMEM((EW,), jnp.int32),
            pltpu.VMEM((GW,), jnp.int32),
            pltpu.VMEM((GW,), jnp.int32),
            pltpu.VMEM((GW, D), jnp.float32),
            pltpu.VMEM((GW, D), jnp.float32),
            pltpu.SemaphoreType.DMA,
        ],
        name="k3b_gather",
    )
    return kern(rowS, colS, ms_tab, v_tab)


def _k3c_unperm(inv_hbm, ue_hbm, out_hbm, invf_v, invj_v, uwin, sem):
    wid = _wid()
    eb = _mo(wid * EW)
    nch = EW // GW
    nv = GW // LANES

    pltpu.sync_copy(inv_hbm.at[pl.ds(eb, EW)], invf_v)

    def body(i, _):
        sl = pl.ds(_mo(i * LANES, LANES), LANES)
        invf_v[sl] = jnp.clip(invf_v[sl], 0, E - 1)
        return 0
    lax.fori_loop(0, EW // LANES, body, 0)

    def win(j, _):
        cb = _mo(j * GW)

        def cp(i2, _):
            sls = pl.ds(cb + _mo(i2 * LANES, LANES), LANES)
            sld = pl.ds(_mo(i2 * LANES, LANES), LANES)
            invj_v[sld] = invf_v[sls]
            return 0
        lax.fori_loop(0, nv, cp, 0)
        pltpu.async_copy(ue_hbm.at[invj_v], uwin, sem).wait()
        pltpu.sync_copy(uwin, out_hbm.at[pl.ds(eb + cb, GW)])
        return 0
    lax.fori_loop(0, nch, win, 0)


def _run_k3c(inv_p, ue_s):
    kern = pl.kernel(
        _k3c_unperm,
        out_type=_f32((EP2, D)),
        mesh=_mesh,
        compiler_params=_SC_PARAMS,
        scratch_types=[
            pltpu.VMEM((EW,), jnp.int32),
            pltpu.VMEM((GW,), jnp.int32),
            pltpu.VMEM((GW, D), jnp.float32),
            pltpu.SemaphoreType.DMA,
        ],
        name="k3c_unpermute",
    )
    return kern(inv_p, ue_s)


W6 = 80


def _k6_sums(ue_hbm, row_hbm, col_hbm, sum_out, sum_in, cnt_in,
             uew, uew2, riw, riw2, ones_w, zb, zc, tab_sh, cnt_sh,
             sem, sem2, sem3, sem4):
    c = lax.axis_index("c")
    s = lax.axis_index("s")
    eb = _mo(s * ECH)
    rows_per_tile = NPAD // NSUB

    nvz = D // LANES

    def zb_fill(i, _):
        zb[i // nvz, pl.ds(_mo((i % nvz) * LANES, LANES), LANES)] = (
            jnp.zeros((LANES,), jnp.float32))
        return 0
    lax.fori_loop(0, 64 * nvz, zb_fill, 0)

    def z(i, _):
        pltpu.sync_copy(
            zb, tab_sh.at[pl.ds(_mo(s * rows_per_tile + i * 64), 64)])
        return 0
    lax.fori_loop(0, rows_per_tile // 64, z, 0)

    def zc_fill(i, _):
        zc[pl.ds(_mo(i * LANES, LANES), LANES)] = jnp.zeros(
            (LANES,), jnp.float32)
        return 0
    lax.fori_loop(0, rows_per_tile // LANES, zc_fill, 0)
    pltpu.sync_copy(zc, cnt_sh.at[pl.ds(_mo(s * rows_per_tile),
                                        rows_per_tile)])

    def ones_fill(i, _):
        ones_w[pl.ds(_mo(i * LANES, LANES), LANES)] = jnp.ones(
            (LANES,), jnp.float32)
        return 0
    lax.fori_loop(0, W6 // LANES, ones_fill, 0)

    plsc.subcore_barrier()

    nwin6 = ECH // W6
    uews = [uew, uew2]
    riws = [riw, riw2]
    semsu = [sem, sem3]
    hs = [None, None]

    for w in range(nwin6):
        slot = w % 2
        base = _mo(eb + w * W6)
        hu = pltpu.async_copy(ue_hbm.at[pl.ds(base, W6)], uews[slot],
                              semsu[slot])

        @pl.when(c == 0)
        def _():
            pltpu.sync_copy(row_hbm.at[pl.ds(base, W6)], riws[slot])

        @pl.when(c == 1)
        def _():
            pltpu.sync_copy(col_hbm.at[pl.ds(base, W6)], riws[slot])

        hs[slot] = hu
        if w > 0:
            ps = (w - 1) % 2
            hs[ps].wait()
            pltpu.sync_copy(uews[ps], tab_sh.at[riws[ps]], add=True)
            pltpu.sync_copy(ones_w, cnt_sh.at[riws[ps]], add=True)
    ps = (nwin6 - 1) % 2
    hs[ps].wait()
    pltpu.sync_copy(uews[ps], tab_sh.at[riws[ps]], add=True)
    pltpu.sync_copy(ones_w, cnt_sh.at[riws[ps]], add=True)

    plsc.subcore_barrier()

    sl = pl.ds(_mo(s * rows_per_tile), rows_per_tile)
    slc = sl

    @pl.when(c == 0)
    def _():
        pltpu.sync_copy(tab_sh.at[sl], sum_out.at[sl])

    @pl.when(c == 1)
    def _():
        pltpu.sync_copy(tab_sh.at[sl], sum_in.at[sl])
        pltpu.sync_copy(cnt_sh.at[slc], cnt_in.at[slc])


def _run_k6(ue, row, col):
    kern = pl.kernel(
        _k6_sums,
        out_type=(_f32((NPAD, D)), _f32((NPAD, D)), _f32((NPAD,))),
        mesh=_mesh,
        compiler_params=_SC_PARAMS,
        scratch_types=[
            pltpu.VMEM((W6, D), jnp.float32),
            pltpu.VMEM((W6, D), jnp.float32),
            pltpu.VMEM((W6,), jnp.int32),
            pltpu.VMEM((W6,), jnp.int32),
            pltpu.VMEM((W6,), jnp.float32),
            pltpu.VMEM((64, D), jnp.float32),
            pltpu.VMEM((NPAD // NSUB,), jnp.float32),
            pltpu.VMEM_SHARED((NPAD, D), jnp.float32),
            pltpu.VMEM_SHARED((NPAD,), jnp.float32),
            pltpu.SemaphoreType.DMA,
            pltpu.SemaphoreType.DMA,
            pltpu.SemaphoreType.DMA,
            pltpu.SemaphoreType.DMA,
        ],
        name="k6_twin_sums",
    )
    return kern(ue, row, col)


def _k0a_pt(cn_ref, cr_ref, wt, bt, out):
    cnf = cn_ref[...]
    crf = cr_ref[...]
    a = jnp.repeat(cnf, 160, axis=0)
    b = jnp.tile(jnp.repeat(crf, 20, axis=0), (20, 1))
    cpart = jnp.tile(cnf, (160, 1))
    te = a + b + cpart
    nrm = jnp.sqrt(jnp.sum(te * te, axis=1, keepdims=True))
    te = te / (nrm + 1e-8)
    out[...] = te @ wt[...] + bt[...]


def _run_k0a(clip_node, clip_rel, wt, bt):
    return pl.pallas_call(
        _k0a_pt,
        out_shape=_f32((NCOMBO, D)),
    )(clip_node, clip_rel, wt, bt.reshape(1, D))


def _k0b_tabs(x_blk, pos_blk, wq, bq, wv, bv, row_tab, col_tab, v_tab):
    x = x_blk[...]
    p16 = pos_blk[...]
    q = x @ wq[...] + bq[...]
    v = x @ wv[...] + bv[...]
    zr = jnp.zeros((x.shape[0], TA - 2 * D - 16), jnp.float32)
    row_tab[...] = jnp.concatenate([q, x, p16, zr], axis=1)
    col_tab[...] = jnp.concatenate([x, v, p16, zr], axis=1)
    v_tab[...] = v


def _run_k0b(x, pos16, wq, bq, wv, bv):
    nb = N // 1000
    return pl.pallas_call(
        _k0b_tabs,
        grid=(nb,),
        in_specs=[
            pl.BlockSpec((1000, D), lambda i: (i, 0)),
            pl.BlockSpec((1000, 16), lambda i: (i, 0)),
            pl.BlockSpec((D, D), lambda i: (0, 0)),
            pl.BlockSpec((1, D), lambda i: (0, 0)),
            pl.BlockSpec((D, D), lambda i: (0, 0)),
            pl.BlockSpec((1, D), lambda i: (0, 0)),
        ],
        out_specs=[
            pl.BlockSpec((1000, TA), lambda i: (i, 0)),
            pl.BlockSpec((1000, TB), lambda i: (i, 0)),
            pl.BlockSpec((1000, D), lambda i: (i, 0)),
        ],
        out_shape=[_f32((N, TA)), _f32((N, TB)), _f32((N, D))],
    )(x, pos16, wq, bq.reshape(1, D), wv, bv.reshape(1, D))


BE = 640


def _k4_edge(a_ref, b_ref, t_ref, r_ref, mf_ref, ef_ref,
             wk, bk, wqk, wkk, wtk, b1k, w2s, b2s,
             dw1a, dw1b, db1, dw2, db2,
             w1a, w1b, w1c, w1d, eub1, euw2, eub2,
             lp_out, ue_out):
    a = a_ref[...]
    b = b_ref[...]
    q = a[:, 0:D]
    xr = a[:, D:2 * D]
    pr = a[:, 2 * D:2 * D + 16]
    xc = b[:, 0:D]
    pc = b[:, 2 * D:2 * D + 16]
    ef = ef_ref[...]
    t = t_ref[...]

    k = ef @ wk[...] + bk[...]
    h1 = jnp.maximum(
        q @ wqk[...] + k @ wkk[...] + t @ wtk[...] + b1k[...], 0.0)
    lg = h1 @ w2s[...] + b2s[...]

    diff = pr - pc
    dist = jnp.sqrt(jnp.sum(diff * diff, axis=1, keepdims=True) + 1e-12)
    hd = jnp.maximum(diff @ dw1a[...] + dist * dw1b[...] + db1[...], 0.0)
    dm = jax.nn.sigmoid(hd @ dw2[...] + db2[...])

    lp_out[...] = jnp.concatenate(
        [lg, dm, jnp.zeros((lg.shape[0], 15), jnp.float32)], axis=1)

    rev = r_ref[...] * mf_ref[...]
    hu = jnp.maximum(
        xr @ w1a[...] + xc @ w1b[...] + ef @ w1c[...] + rev @ w1d[...]
        + eub1[...], 0.0)
    ue_out[...] = hu @ euw2[...] + eub2[...]


def _run_k4(a, b, t, r, mf, efs, wk, bk, wqk, wkk, wtk, b1k, w2s, b2s,
            dw1a, dw1b, db1, dw2, db2, w1a, w1b, w1c, w1d, eub1, euw2, eub2):
    nb = E // BE
    full = lambda shape: pl.BlockSpec(shape, lambda i: (0, 0))
    return pl.pallas_call(
        _k4_edge,
        grid=(nb,),
        in_specs=[
            pl.BlockSpec((BE, TA), lambda i: (i, 0)),
            pl.BlockSpec((BE, TB), lambda i: (i, 0)),
            pl.BlockSpec((BE, D), lambda i: (i, 0)),
            pl.BlockSpec((BE, D), lambda i: (i, 0)),
            pl.BlockSpec((BE, 1), lambda i: (i, 0)),
            pl.BlockSpec((BE, D), lambda i: (i, 0)),
            full((D, D)), full((1, D)),
            full((D, 640)), full((D, 640)), full((D, 640)),
            full((1, 640)), full((640, 16)), full((1, 16)),
            full((16, 32)), full((1, 32)), full((1, 32)),
            full((32, 1)), full((1, 1)),
            full((D, 384)), full((D, 384)), full((D, 384)), full((D, 384)),
            full((1, 384)), full((384, D)), full((1, D)),
        ],
        out_specs=[
            pl.BlockSpec((BE, 32), lambda i: (i, 0)),
            pl.BlockSpec((BE, D), lambda i: (i, 0)),
        ],
        out_shape=[_f32((E, 32)), _f32((E, D))],
    )(a, b, t, r, mf, efs, wk, bk, wqk, wkk, wtk, b1k, w2s, b2s,
      dw1a, dw1b, db1, dw2, db2, w1a, w1b, w1c, w1d, eub1, euw2, eub2)


def _k4b_msg(lp_ref, msr_ref, vc_ref, selm, sels, prc, msg_out):
    lp = lp_ref[...]
    lg = lp[:, 0:16]
    dm = lp[:, 16:17]
    msr = msr_ref[...]
    m = msr @ selm[...]
    s = msr @ sels[...]
    p = jnp.exp(lg - m) / (s + 1e-9)
    alpha = (p @ prc[...]) * dm
    msg_out[...] = vc_ref[...] * alpha


def _run_k4b(lp, msr, vc, selm, sels, prc):
    nb = E // BE
    full = lambda shape: pl.BlockSpec(shape, lambda i: (0, 0))
    return pl.pallas_call(
        _k4b_msg,
        grid=(nb,),
        in_specs=[
            pl.BlockSpec((BE, 32), lambda i: (i, 0)),
            pl.BlockSpec((BE, D), lambda i: (i, 0)),
            pl.BlockSpec((BE, D), lambda i: (i, 0)),
            full((D, 16)), full((D, 16)), full((16, D)),
        ],
        out_specs=pl.BlockSpec((BE, D), lambda i: (i, 0)),
        out_shape=_f32((E, D)),
    )(lp, msr, vc, selm, sels, prc)


def _k7_node(x_ref, agg_ref, so_ref, si_ref, co_ref, ci_ref,
             nw1a, nw1b, nb1, nw2, nb2, eawa, eawb, eab, out):
    x = x_ref[...]
    agg = agg_ref[...]
    h = jnp.maximum(x @ nw1a[...] + agg @ nw1b[...] + nb1[...], 0.0)
    un = h @ nw2[...] + nb2[...]
    co = jnp.maximum(co_ref[...], 1.0)
    ci = jnp.maximum(ci_ref[...], 1.0)
    om = so_ref[...] / co
    im = si_ref[...] / ci
    gate = jax.nn.sigmoid(om @ eawa[...] + im @ eawb[...] + eab[...])
    out[...] = un * gate


def _run_k7(x, agg, so, si, co, ci, nw1a, nw1b, nb1, nw2, nb2,
            eawa, eawb, eab):
    nb = N // 1000
    full = lambda shape: pl.BlockSpec(shape, lambda i: (0, 0))
    return pl.pallas_call(
        _k7_node,
        grid=(nb,),
        in_specs=[
            pl.BlockSpec((1000, D), lambda i: (i, 0)),
            pl.BlockSpec((1000, D), lambda i: (i, 0)),
            pl.BlockSpec((1000, D), lambda i: (i, 0)),
            pl.BlockSpec((1000, D), lambda i: (i, 0)),
            pl.BlockSpec((1000, 1), lambda i: (i, 0)),
            pl.BlockSpec((1000, 1), lambda i: (i, 0)),
            full((D, 256)), full((D, 256)), full((1, 256)),
            full((256, D)), full((1, D)),
            full((D, D)), full((D, D)), full((1, D)),
        ],
        out_specs=pl.BlockSpec((1000, D), lambda i: (i, 0)),
        out_shape=_f32((N, D)),
    )(x, agg, so, si, co, ci, nw1a, nw1b, nb1, nw2, nb2, eawa, eawb, eab)


def kernel(x, edge_feature, node_positions, params, edge_index,
           gt_rel_label, gt_obj_label):
    p = params
    row = edge_index[0].astype(jnp.int32)
    col = edge_index[1].astype(jnp.int32)
    obj = gt_obj_label.astype(jnp.int32)
    rel = gt_rel_label.astype(jnp.int32)
    pos16 = jnp.pad(node_positions, ((0, 0), (0, 13)))

    eye = jnp.eye(H, dtype=jnp.float32)
    wqk = jnp.concatenate([jnp.kron(eye, p['a3W1'][:DH]),
                           jnp.kron(eye, p['atW1'][:DH])], axis=1)
    wkk = jnp.concatenate([jnp.kron(eye, p['a3W1'][DH:2 * DH]),
                           jnp.kron(eye, p['atW1'][DH:2 * DH])], axis=1)
    wtk = jnp.concatenate([jnp.zeros((D, 256), jnp.float32),
                           jnp.kron(eye, p['atW1'][2 * DH:3 * DH])], axis=1)
    b1k = jnp.concatenate([jnp.tile(p['a3b1'], H),
                           jnp.tile(p['atb1'], H)]).reshape(1, 640)
    w2s_a = jnp.kron(eye, p['a3W2'].sum(axis=1)[:, None])
    w2s_t = jnp.kron(eye, p['atW2'].sum(axis=1)[:, None])
    w2s = jnp.concatenate([
        jnp.concatenate([w2s_a, jnp.zeros((256, 8), jnp.float32)], axis=1),
        jnp.concatenate([jnp.zeros((384, 8), jnp.float32), w2s_t], axis=1),
    ], axis=0) / TEMP
    b2s = jnp.concatenate([
        jnp.full((8,), p['a3b2'].sum(), jnp.float32),
        jnp.full((8,), p['atb2'].sum(), jnp.float32)]).reshape(1, 16) / TEMP

    dw1a = jnp.pad(p['dW1'][:3], ((0, 13), (0, 0)))
    dw1b = p['dW1'][3].reshape(1, 32)
    db1 = p['db1'].reshape(1, 32)
    dw2 = p['dW2']
    db2 = p['db2'].reshape(1, 1)

    w1a = p['euW1'][0:D]
    w1b = p['euW1'][D:2 * D]
    w1c = p['euW1'][2 * D:3 * D]
    w1d = p['euW1'][3 * D:4 * D]
    eub1 = p['eub1'].reshape(1, 384)
    eub2 = p['eub2'].reshape(1, D)

    nw1a = p['nuW1'][0:D]
    nw1b = p['nuW1'][D:2 * D]
    nb1 = p['nub1'].reshape(1, 256)
    nb2 = p['nub2'].reshape(1, D)
    eawa = p['eaW'][0:D]
    eawb = p['eaW'][D:2 * D]
    eab = p['eab'].reshape(1, D)

    selm = jnp.zeros((D, 16), jnp.float32).at[:16, :].set(jnp.eye(16))
    sels = jnp.zeros((D, 16), jnp.float32).at[16:32, :].set(jnp.eye(16))
    prc = jnp.concatenate([jnp.kron(eye, jnp.ones((1, DH), jnp.float32)),
                           jnp.kron(eye, jnp.ones((1, DH), jnp.float32))],
                          axis=0) * 0.5

    pt_tab = _run_k0a(p['clip_node'], p['clip_rel'], p['Wt'], p['bt'])
    row_tab, col_tab, v_tab = _run_k0b(x, pos16, p['Wq'], p['bq'],
                                       p['Wv'], p['bv'])
    rowS, colS, idxS, cnt, off, inv2d = _run_k1(row, col)
    ridxS, matchS = _run_k2(rowS, colS, off, cnt, idxS)
    a_g, b_g, t_g, r_g, efs = _run_k3(rowS, colS, idxS, obj, rel, ridxS,
                                      row_tab, col_tab, pt_tab, edge_feature)

    mf = matchS.astype(jnp.float32).reshape(EP2, 1)
    lp, ue_s = _run_k4(a_g, b_g, t_g, r_g, mf, efs,
                       p['Wk'], p['bk'].reshape(1, D), wqk, wkk, wtk, b1k,
                       w2s, b2s, dw1a, dw1b, db1, dw2, db2,
                       w1a, w1b, w1c, w1d, eub1, p['euW2'], eub2)

    ms_tab = _run_k5a(lp.reshape(E * 32), rowS, off).reshape(NPAD, D)
    msr, vc = _run_k3b(rowS, colS, ms_tab, v_tab)
    msg = _run_k4b(lp, msr, vc, selm, sels, prc)
    agg = _run_k5b(msg.reshape(E * D), rowS, off).reshape(NPAD, D)

    inv_p = jnp.pad(inv2d[:E], (0, EP2 - E))
    ue = _run_k3c(inv_p, ue_s)[:E]

    sum_out, sum_in, cnt_in = _run_k6(ue, row, col)

    updated_node = _run_k7(
        x, agg[:N], sum_out[:N], sum_in[:N],
        cnt[:N].astype(jnp.float32).reshape(N, 1), cnt_in[:N].reshape(N, 1),
        nw1a, nw1b, nb1, p['nuW2'], nb2, eawa, eawb, eab)

    return updated_node, ue
